# baseline (speedup 1.0000x reference)
import functools

import jax
import jax.numpy as jnp
from jax import lax
from jax.experimental import pallas as pl
from jax.experimental.pallas import tpu as pltpu
from jax.experimental.pallas import tpu_sc as plsc

N = 10000
E = 320000
IN = 128
HEADS = 2
R = 4
HID = 128
OUTS = 33
CPS = 128
NP = 10240
TN = 256
NT = NP // TN

NC = 2
NS = 16
NW = NC * NS
ETP = 10240
EP = ETP * NW
CH = 128
NCH = ETP // CH
NU = 10240
URT = NU // NS
DCH = NU // NS


@functools.lru_cache(maxsize=None)
def _make_pre(h_, r_, cp):
    def body(x_ref, w_ref, q_ref, k_ref, xr_ref, qt_ref, kt_ref, qm_ref, km_ref):
        xr = jnp.dot(x_ref[...], w_ref[0, 0], preferred_element_type=jnp.float32)
        xr_ref[0, 0] = xr
        qv = jnp.sum(xr * q_ref[0, 0, 0][None, :], axis=1)
        kv = jnp.sum(xr * k_ref[0, 0, 0][None, :], axis=1)
        qt_ref[0, 0] = qv
        kt_ref[0, 0] = kv
        qm_ref[0, 0] = jnp.full((TN,), jnp.max(qv), jnp.float32)
        km_ref[0, 0] = jnp.full((TN,), jnp.max(kv), jnp.float32)

    g = h_ * r_ * NT
    return pl.pallas_call(
        body,
        grid=(h_, r_, NT),
        in_specs=[
            pl.BlockSpec((TN, IN), lambda h, r, i: (i, 0)),
            pl.BlockSpec((1, 1, IN, cp), lambda h, r, i: (h, r, 0, 0)),
            pl.BlockSpec((1, 1, 1, cp), lambda h, r, i: (h, r, 0, 0)),
            pl.BlockSpec((1, 1, 1, cp), lambda h, r, i: (h, r, 0, 0)),
        ],
        out_specs=[
            pl.BlockSpec((1, 1, TN, cp), lambda h, r, i: (h, r, i, 0)),
            pl.BlockSpec((1, 1, TN), lambda h, r, i: ((h * r_ + r) * NT + i, 0, 0)),
            pl.BlockSpec((1, 1, TN), lambda h, r, i: ((h * r_ + r) * NT + i, 0, 0)),
            pl.BlockSpec((1, 1, TN), lambda h, r, i: ((h * r_ + r) * NT + i, 0, 0)),
            pl.BlockSpec((1, 1, TN), lambda h, r, i: ((h * r_ + r) * NT + i, 0, 0)),
        ],
        out_shape=[
            jax.ShapeDtypeStruct((h_, r_, NP, cp), jnp.float32),
            jax.ShapeDtypeStruct((g, 1, TN), jnp.float32),
            jax.ShapeDtypeStruct((g, 1, TN), jnp.float32),
            jax.ShapeDtypeStruct((g, 1, TN), jnp.float32),
            jax.ShapeDtypeStruct((g, 1, TN), jnp.float32),
        ],
    )


@functools.lru_cache(maxsize=None)
def _make_edge(tbl, cp, tstride):
    mesh = plsc.VectorSubcoreMesh(core_axis_name="c", subcore_axis_name="s")
    stg = tbl // NS
    zr = 128
    nvec = cp // 16

    def body(qtab_h, ktab_h, src_h, dst_h, typ_h, xr_h, m_h,
             u0, u1, d0, d1,
             src_v, dst_v, typ_v, gsrc_v, gdst_v, q_v, k_v, ex_v, rows_v,
             m_v, zb_v, zd_v, u_s, den_s, qtab_s, ktab_s, sem0):
        cid = lax.axis_index("c")
        sid = lax.axis_index("s")
        wid = cid * NS + sid

        pltpu.sync_copy(qtab_h.at[pl.ds(sid * stg, stg)], qtab_s.at[pl.ds(sid * stg, stg)])
        pltpu.sync_copy(ktab_h.at[pl.ds(sid * stg, stg)], ktab_s.at[pl.ds(sid * stg, stg)])
        pltpu.sync_copy(m_h, m_v)

        def zrow(i, c):
            for j in range(nvec):
                zb_v[i, pl.ds(j * 16, 16)] = jnp.zeros((16,), jnp.float32)
            return c
        lax.fori_loop(0, zr, zrow, 0)

        def zden(i, c):
            zd_v[pl.ds(i * 16, 16)] = jnp.zeros((16,), jnp.float32)
            return c
        lax.fori_loop(0, DCH // 16, zden, 0)

        for j in range(URT // zr):
            pltpu.sync_copy(zb_v, u_s.at[pl.ds(sid * URT + j * zr, zr)])

        pltpu.sync_copy(zd_v, den_s.at[pl.ds(sid * DCH, DCH)])

        plsc.subcore_barrier()

        mv = m_v[...]

        def chunk(c, carry):
            base = wid * ETP + c * CH
            pltpu.sync_copy(src_h.at[pl.ds(base, CH)], src_v)
            pltpu.sync_copy(dst_h.at[pl.ds(base, CH)], dst_v)
            pltpu.sync_copy(typ_h.at[pl.ds(base, CH)], typ_v)
            for i in range(CH // 16):
                sl = pl.ds(i * 16, 16)
                tv = typ_v[sl]
                gsrc_v[sl] = tv * tstride + src_v[sl]
                gdst_v[sl] = tv * tstride + dst_v[sl]
            rcp = pltpu.async_copy(xr_h.at[gsrc_v], rows_v, sem0)
            pltpu.sync_copy(qtab_s.at[gdst_v], q_v)
            pltpu.sync_copy(ktab_s.at[gsrc_v], k_v)
            for i in range(CH // 16):
                sl = pl.ds(i * 16, 16)
                z = q_v[sl] + k_v[sl]
                z = jnp.maximum(z, 0.2 * z)
                ex_v[sl] = jnp.exp(z - mv)
            pltpu.sync_copy(ex_v, den_s.at[dst_v], add=True)
            rcp.wait()

            def scale(g, c2):
                exg = ex_v[pl.ds(g * 16, 16)]
                for i in range(16):
                    s = exg[i]
                    for j in range(nvec):
                        slj = pl.ds(j * 16, 16)
                        rows_v[g * 16 + i, slj] = rows_v[g * 16 + i, slj] * s
                return c2
            lax.fori_loop(0, CH // 16, scale, 0)
            pltpu.sync_copy(rows_v, u_s.at[dst_v], add=True)
            return carry

        lax.fori_loop(0, NCH, chunk, 0)
        plsc.subcore_barrier()

        @pl.when(cid == 0)
        def _():
            pltpu.sync_copy(u_s.at[pl.ds(sid * URT, URT)], u0.at[pl.ds(sid * URT, URT)])

        @pl.when(cid == 1)
        def _():
            pltpu.sync_copy(u_s.at[pl.ds(sid * URT, URT)], u1.at[pl.ds(sid * URT, URT)])

        @pl.when(cid == 0)
        def _():
            pltpu.sync_copy(den_s.at[pl.ds(sid * DCH, DCH)], d0.at[pl.ds(sid * DCH, DCH)])

        @pl.when(cid == 1)
        def _():
            pltpu.sync_copy(den_s.at[pl.ds(sid * DCH, DCH)], d1.at[pl.ds(sid * DCH, DCH)])

    return pl.kernel(
        body,
        out_type=(
            jax.ShapeDtypeStruct((NU, cp), jnp.float32),
            jax.ShapeDtypeStruct((NU, cp), jnp.float32),
            jax.ShapeDtypeStruct((NU,), jnp.float32),
            jax.ShapeDtypeStruct((NU,), jnp.float32),
        ),
        mesh=mesh,
        scratch_types=[
            pltpu.VMEM((CH,), jnp.int32),
            pltpu.VMEM((CH,), jnp.int32),
            pltpu.VMEM((CH,), jnp.int32),
            pltpu.VMEM((CH,), jnp.int32),
            pltpu.VMEM((CH,), jnp.int32),
            pltpu.VMEM((CH,), jnp.float32),
            pltpu.VMEM((CH,), jnp.float32),
            pltpu.VMEM((CH,), jnp.float32),
            pltpu.VMEM((CH, cp), jnp.float32),
            pltpu.VMEM((16,), jnp.float32),
            pltpu.VMEM((zr, cp), jnp.float32),
            pltpu.VMEM((DCH,), jnp.float32),
            pltpu.VMEM_SHARED((NU, cp), jnp.float32),
            pltpu.VMEM_SHARED((NU,), jnp.float32),
            pltpu.VMEM_SHARED((tbl,), jnp.float32),
            pltpu.VMEM_SHARED((tbl,), jnp.float32),
            pltpu.SemaphoreType.DMA,
        ],
    )


RB = 400
NB = N // RB


@functools.lru_cache(maxsize=None)
def _make_post_rgat(cp):
    def body(u00, u01, u10, u11, d00, d01, d10, d11, b_ref, o_ref):
        den0 = d00[0, 0] + d01[0, 0] + 1e-16
        den1 = d10[0, 0] + d11[0, 0] + 1e-16
        num0 = u00[...] + u01[...]
        num1 = u10[...] + u11[...]
        o = 0.5 * (num0 / den0[:, None] + num1 / den1[:, None]) + b_ref[0][None, :]
        o_ref[...] = jnp.where(o > 0, o, jnp.exp(o) - 1.0)

    ub = pl.BlockSpec((RB, cp), lambda i: (i, 0))
    db = pl.BlockSpec((1, 1, RB), lambda i: (i, 0, 0))
    return pl.pallas_call(
        body,
        grid=(NB,),
        in_specs=[ub, ub, ub, ub, db, db, db, db,
                  pl.BlockSpec((1, cp), lambda i: (0, 0))],
        out_specs=ub,
        out_shape=jax.ShapeDtypeStruct((N, cp), jnp.float32),
    )


@functools.lru_cache(maxsize=None)
def _make_post_self(cp):
    def body(u0, u1, d0, d1, b_ref, o_ref):
        den = d0[0, 0] + d1[0, 0] + 1e-16
        num = u0[...] + u1[...]
        o = num / den[:, None] + b_ref[0][None, :]
        o_ref[...] = jnp.where(o > 0, o, jnp.exp(o) - 1.0)

    ub = pl.BlockSpec((RB, cp), lambda i: (i, 0))
    db = pl.BlockSpec((1, 1, RB), lambda i: (i, 0, 0))
    return pl.pallas_call(
        body,
        grid=(NB,),
        in_specs=[ub, ub, db, db, pl.BlockSpec((1, cp), lambda i: (0, 0))],
        out_specs=ub,
        out_shape=jax.ShapeDtypeStruct((N, cp), jnp.float32),
    )


def _leaky(z):
    return jnp.maximum(z, 0.2 * z)


def _pad_rows(a):
    return jnp.pad(a, ((0, NP - a.shape[0]), (0, 0)))


def _prep_w(w, q, k, heads, outc, cp):
    d = w.shape[1]
    wp = w.reshape(R, d, heads, outc).transpose(2, 0, 1, 3)
    qp = q.reshape(R, heads, outc).transpose(1, 0, 2)[:, :, None, :]
    kp = k.reshape(R, heads, outc).transpose(1, 0, 2)[:, :, None, :]
    if cp != outc:
        pad = ((0, 0), (0, 0), (0, 0), (0, cp - outc))
        wp = jnp.pad(wp, pad)
        qp = jnp.pad(qp, pad)
        kp = jnp.pad(kp, pad)
    return wp, qp, kp


def _rgat(xp, w, q, k, b, src, dst, typ, outc, cp):
    wp, qp, kp = _prep_w(w, q, k, HEADS, outc, cp)
    xr, qt, kt, qm, km = _make_pre(HEADS, R, cp)(xp, wp, qp, kp)
    xrh = xr.reshape(HEADS, R * NP, cp)
    qtab = qt.reshape(HEADS, R * NP)
    ktab = kt.reshape(HEADS, R * NP)
    mh = _leaky(qm.reshape(HEADS, -1).max(axis=1) + km.reshape(HEADS, -1).max(axis=1))
    edge = _make_edge(R * NP, cp, NP)
    us, ds_ = [], []
    for h in range(HEADS):
        m16 = jnp.broadcast_to(jnp.reshape(mh[h], (1,)), (16,))
        u0, u1, d0, d1 = edge(qtab[h], ktab[h], src, dst, typ, xrh[h], m16)
        us += [u0, u1]
        ds_ += [d0[:N].reshape(NB, 1, RB), d1[:N].reshape(NB, 1, RB)]
    bp = jnp.pad(b, (0, cp - b.shape[0]))[None, :]
    return _make_post_rgat(cp)(us[0], us[1], us[2], us[3],
                               ds_[0], ds_[1], ds_[2], ds_[3], bp)


def kernel(x, edge_index, edge_type, W1, Q1, K1, b1, WS, QS, KS, bS,
           W2, Q2, K2, b2, WT, QT, KT, bT, Wr, a_src, a_dst, br):
    pad = EP - E
    src = jnp.concatenate([edge_index[0], jnp.zeros((pad,), jnp.int32)])
    dst = jnp.concatenate([edge_index[1], jnp.full((pad,), N, jnp.int32)])
    typ = jnp.concatenate([edge_type, jnp.zeros((pad,), jnp.int32)])
    xp = _pad_rows(x)

    x_s = _rgat(xp, W1, Q1, K1, b1, src, dst, typ, HID, HID)
    x_in = _rgat(_pad_rows(x_s), WS, QS, KS, bS, src, dst, typ, OUTS, CPS)[:, :OUTS]

    x_t = _rgat(xp, W2, Q2, K2, b2, src, dst, typ, HID, HID)
    x_out = _rgat(_pad_rows(x_t), WT, QT, KT, bT, src, dst, typ, OUTS, CPS)[:, :OUTS]

    wr = Wr[None, None]
    qp = a_dst[None, None, None, :]
    kp = a_src[None, None, None, :]
    xr, qt, kt, qm, km = _make_pre(1, 1, HID)(xp, wr, qp, kp)
    m0 = _leaky(qm.max() + km.max())
    m16 = jnp.broadcast_to(jnp.reshape(m0, (1,)), (16,))
    u0, u1, d0, d1 = _make_edge(NP, HID, 0)(
        qt.reshape(NP), kt.reshape(NP), src, dst, typ, xr.reshape(NP, HID), m16)
    x_self = _make_post_self(HID)(u0, u1, d0[:N].reshape(NB, 1, RB), d1[:N].reshape(NB, 1, RB),
                                  br[None, :])
    return (x_in, x_out, x_self)


# speedup vs baseline: 1.9506x; 1.9506x over previous
import functools

import jax
import jax.numpy as jnp
from jax import lax
from jax.experimental import pallas as pl
from jax.experimental.pallas import tpu as pltpu
from jax.experimental.pallas import tpu_sc as plsc

N = 10000
E = 320000
IN = 128
HEADS = 2
R = 4
HID = 128
OUTS = 33
CPS = 128
NP = 10240
TN = 256
NT = NP // TN

NC = 2
NS = 16
NW = NC * NS
ET = E // NW
CH = 80
NCH = ET // CH
NU = 10240
URT = NU // NS
DCH = NU // NS


@functools.lru_cache(maxsize=None)
def _make_pre(h_, r_, cp):
    def body(x_ref, w_ref, q_ref, k_ref, xr_ref, qt_ref, kt_ref, qm_ref, km_ref):
        xr = jnp.dot(x_ref[...], w_ref[0, 0], preferred_element_type=jnp.float32)
        xr_ref[0, 0] = xr
        qv = jnp.sum(xr * q_ref[0, 0, 0][None, :], axis=1)
        kv = jnp.sum(xr * k_ref[0, 0, 0][None, :], axis=1)
        qt_ref[0, 0] = qv
        kt_ref[0, 0] = kv
        qm_ref[0, 0] = jnp.full((TN,), jnp.max(qv), jnp.float32)
        km_ref[0, 0] = jnp.full((TN,), jnp.max(kv), jnp.float32)

    g = h_ * r_ * NT
    return pl.pallas_call(
        body,
        grid=(h_, r_, NT),
        in_specs=[
            pl.BlockSpec((TN, IN), lambda h, r, i: (i, 0)),
            pl.BlockSpec((1, 1, IN, cp), lambda h, r, i: (h, r, 0, 0)),
            pl.BlockSpec((1, 1, 1, cp), lambda h, r, i: (h, r, 0, 0)),
            pl.BlockSpec((1, 1, 1, cp), lambda h, r, i: (h, r, 0, 0)),
        ],
        out_specs=[
            pl.BlockSpec((1, 1, TN, cp), lambda h, r, i: (h, r, i, 0)),
            pl.BlockSpec((1, 1, TN), lambda h, r, i: ((h * r_ + r) * NT + i, 0, 0)),
            pl.BlockSpec((1, 1, TN), lambda h, r, i: ((h * r_ + r) * NT + i, 0, 0)),
            pl.BlockSpec((1, 1, TN), lambda h, r, i: ((h * r_ + r) * NT + i, 0, 0)),
            pl.BlockSpec((1, 1, TN), lambda h, r, i: ((h * r_ + r) * NT + i, 0, 0)),
        ],
        out_shape=[
            jax.ShapeDtypeStruct((h_, r_, NP, cp), jnp.float32),
            jax.ShapeDtypeStruct((g, 1, TN), jnp.float32),
            jax.ShapeDtypeStruct((g, 1, TN), jnp.float32),
            jax.ShapeDtypeStruct((g, 1, TN), jnp.float32),
            jax.ShapeDtypeStruct((g, 1, TN), jnp.float32),
        ],
    )


@functools.lru_cache(maxsize=None)
def _make_edge(tbl, cp, tstride):
    mesh = plsc.VectorSubcoreMesh(core_axis_name="c", subcore_axis_name="s")
    stg = tbl // NS
    zr = 128
    nvec = cp // 16

    def body(qtab_h, ktab_h, src_h, dst_h, typ_h, xr_h, m_h,
             u0, u1, d0, d1,
             src_v, dst_v, typ_v, gsrc_v, gdst_v, q_v, k_v, ex_v, rows_v,
             m_v, zb_v, zd_v, u_s, den_s, qtab_s, ktab_s, sem0):
        cid = lax.axis_index("c")
        sid = lax.axis_index("s")
        wid = cid * NS + sid

        pltpu.sync_copy(qtab_h.at[pl.ds(sid * stg, stg)], qtab_s.at[pl.ds(sid * stg, stg)])
        pltpu.sync_copy(ktab_h.at[pl.ds(sid * stg, stg)], ktab_s.at[pl.ds(sid * stg, stg)])
        pltpu.sync_copy(m_h, m_v)

        def zrow(i, c):
            for j in range(nvec):
                zb_v[i, pl.ds(j * 16, 16)] = jnp.zeros((16,), jnp.float32)
            return c
        lax.fori_loop(0, zr, zrow, 0)

        def zden(i, c):
            zd_v[pl.ds(i * 16, 16)] = jnp.zeros((16,), jnp.float32)
            return c
        lax.fori_loop(0, DCH // 16, zden, 0)

        for j in range(URT // zr):
            pltpu.sync_copy(zb_v, u_s.at[pl.ds(sid * URT + j * zr, zr)])

        pltpu.sync_copy(zd_v, den_s.at[pl.ds(sid * DCH, DCH)])

        plsc.subcore_barrier()

        mv = m_v[...]

        def chunk(c, carry):
            base = wid * ET + c * CH
            pltpu.sync_copy(src_h.at[pl.ds(base, CH)], src_v)
            pltpu.sync_copy(dst_h.at[pl.ds(base, CH)], dst_v)
            pltpu.sync_copy(typ_h.at[pl.ds(base, CH)], typ_v)
            for i in range(CH // 16):
                sl = pl.ds(i * 16, 16)
                tv = typ_v[sl]
                gsrc_v[sl] = tv * tstride + src_v[sl]
                gdst_v[sl] = tv * tstride + dst_v[sl]
            rcp = pltpu.async_copy(xr_h.at[gsrc_v], rows_v, sem0)
            pltpu.sync_copy(qtab_s.at[gdst_v], q_v)
            pltpu.sync_copy(ktab_s.at[gsrc_v], k_v)
            for i in range(CH // 16):
                sl = pl.ds(i * 16, 16)
                z = q_v[sl] + k_v[sl]
                z = jnp.maximum(z, 0.2 * z)
                ex_v[sl] = jnp.exp(z - mv)
            pltpu.sync_copy(ex_v, den_s.at[dst_v], add=True)
            rcp.wait()

            def scale(g, c2):
                exg = ex_v[pl.ds(g * 16, 16)]
                for i in range(16):
                    s = exg[i]
                    for j in range(nvec):
                        slj = pl.ds(j * 16, 16)
                        rows_v[g * 16 + i, slj] = rows_v[g * 16 + i, slj] * s
                return c2
            lax.fori_loop(0, CH // 16, scale, 0)
            pltpu.sync_copy(rows_v, u_s.at[dst_v], add=True)
            return carry

        lax.fori_loop(0, NCH, chunk, 0)
        plsc.subcore_barrier()

        @pl.when(cid == 0)
        def _():
            pltpu.sync_copy(u_s.at[pl.ds(sid * URT, URT)], u0.at[pl.ds(sid * URT, URT)])

        @pl.when(cid == 1)
        def _():
            pltpu.sync_copy(u_s.at[pl.ds(sid * URT, URT)], u1.at[pl.ds(sid * URT, URT)])

        @pl.when(cid == 0)
        def _():
            pltpu.sync_copy(den_s.at[pl.ds(sid * DCH, DCH)], d0.at[pl.ds(sid * DCH, DCH)])

        @pl.when(cid == 1)
        def _():
            pltpu.sync_copy(den_s.at[pl.ds(sid * DCH, DCH)], d1.at[pl.ds(sid * DCH, DCH)])

    return pl.kernel(
        body,
        out_type=(
            jax.ShapeDtypeStruct((NU, cp), jnp.float32),
            jax.ShapeDtypeStruct((NU, cp), jnp.float32),
            jax.ShapeDtypeStruct((NU,), jnp.float32),
            jax.ShapeDtypeStruct((NU,), jnp.float32),
        ),
        mesh=mesh,
        scratch_types=[
            pltpu.VMEM((CH,), jnp.int32),
            pltpu.VMEM((CH,), jnp.int32),
            pltpu.VMEM((CH,), jnp.int32),
            pltpu.VMEM((CH,), jnp.int32),
            pltpu.VMEM((CH,), jnp.int32),
            pltpu.VMEM((CH,), jnp.float32),
            pltpu.VMEM((CH,), jnp.float32),
            pltpu.VMEM((CH,), jnp.float32),
            pltpu.VMEM((CH, cp), jnp.float32),
            pltpu.VMEM((16,), jnp.float32),
            pltpu.VMEM((zr, cp), jnp.float32),
            pltpu.VMEM((DCH,), jnp.float32),
            pltpu.VMEM_SHARED((NU, cp), jnp.float32),
            pltpu.VMEM_SHARED((NU,), jnp.float32),
            pltpu.VMEM_SHARED((tbl,), jnp.float32),
            pltpu.VMEM_SHARED((tbl,), jnp.float32),
            pltpu.SemaphoreType.DMA,
        ],
    )


@functools.lru_cache(maxsize=None)
def _make_edge2(tbl, tstride):
    cp = 128
    mesh = plsc.VectorSubcoreMesh(core_axis_name="c", subcore_axis_name="s")
    stg = tbl // NS
    zr = 64
    nvec = cp // 16

    def body(qt0_h, qt1_h, kt0_h, kt1_h, src_h, dst_h, typ_h, xr_h, m0_h, m1_h,
             u0, u1, d00, d01, d10, d11,
             src_v, dst_v, typ_v, gsrc_v, gdst_v, q0_v, q1_v, k0_v, k1_v,
             ex0_v, ex1_v, rows_v, m0_v, m1_v, zb_v, zd_v,
             u_s, den0_s, den1_s, qt0_s, qt1_s, kt0_s, kt1_s, sem0):
        cid = lax.axis_index("c")
        sid = lax.axis_index("s")
        wid = cid * NS + sid

        sls = pl.ds(sid * stg, stg)
        pltpu.sync_copy(qt0_h.at[sls], qt0_s.at[sls])
        pltpu.sync_copy(qt1_h.at[sls], qt1_s.at[sls])
        pltpu.sync_copy(kt0_h.at[sls], kt0_s.at[sls])
        pltpu.sync_copy(kt1_h.at[sls], kt1_s.at[sls])
        pltpu.sync_copy(m0_h, m0_v)
        pltpu.sync_copy(m1_h, m1_v)

        def zrow(i, c):
            for j in range(nvec):
                zb_v[i, pl.ds(j * 16, 16)] = jnp.zeros((16,), jnp.float32)
            return c
        lax.fori_loop(0, zr, zrow, 0)

        def zden(i, c):
            zd_v[pl.ds(i * 16, 16)] = jnp.zeros((16,), jnp.float32)
            return c
        lax.fori_loop(0, DCH // 16, zden, 0)

        for j in range(URT // zr):
            pltpu.sync_copy(zb_v, u_s.at[pl.ds(sid * URT + j * zr, zr)])
        pltpu.sync_copy(zd_v, den0_s.at[pl.ds(sid * DCH, DCH)])
        pltpu.sync_copy(zd_v, den1_s.at[pl.ds(sid * DCH, DCH)])

        plsc.subcore_barrier()
        mv0 = m0_v[...]
        mv1 = m1_v[...]

        def chunk(c, carry):
            base = wid * ET + c * CH
            pltpu.sync_copy(src_h.at[pl.ds(base, CH)], src_v)
            pltpu.sync_copy(dst_h.at[pl.ds(base, CH)], dst_v)
            pltpu.sync_copy(typ_h.at[pl.ds(base, CH)], typ_v)
            for i in range(CH // 16):
                sl = pl.ds(i * 16, 16)
                tv = typ_v[sl]
                gsrc_v[sl] = tv * tstride + src_v[sl]
                gdst_v[sl] = tv * tstride + dst_v[sl]
            rcp = pltpu.async_copy(xr_h.at[gsrc_v], rows_v, sem0)
            pltpu.sync_copy(qt0_s.at[gdst_v], q0_v)
            pltpu.sync_copy(qt1_s.at[gdst_v], q1_v)
            pltpu.sync_copy(kt0_s.at[gsrc_v], k0_v)
            pltpu.sync_copy(kt1_s.at[gsrc_v], k1_v)
            for i in range(CH // 16):
                sl = pl.ds(i * 16, 16)
                z0 = q0_v[sl] + k0_v[sl]
                z0 = jnp.maximum(z0, 0.2 * z0)
                ex0_v[sl] = jnp.exp(z0 - mv0)
                z1 = q1_v[sl] + k1_v[sl]
                z1 = jnp.maximum(z1, 0.2 * z1)
                ex1_v[sl] = jnp.exp(z1 - mv1)
            pltpu.sync_copy(ex0_v, den0_s.at[dst_v], add=True)
            pltpu.sync_copy(ex1_v, den1_s.at[dst_v], add=True)
            rcp.wait()

            def scale(g, c2):
                exg0 = ex0_v[pl.ds(g * 16, 16)]
                exg1 = ex1_v[pl.ds(g * 16, 16)]
                for i in range(16):
                    s0 = exg0[i]
                    s1 = exg1[i]
                    for j in range(3):
                        slj = pl.ds(j * 16, 16)
                        rows_v[g * 16 + i, slj] = rows_v[g * 16 + i, slj] * s0
                    for j in range(3, 6):
                        slj = pl.ds(j * 16, 16)
                        rows_v[g * 16 + i, slj] = rows_v[g * 16 + i, slj] * s1
                return c2
            lax.fori_loop(0, CH // 16, scale, 0)
            pltpu.sync_copy(rows_v, u_s.at[dst_v], add=True)
            return carry

        lax.fori_loop(0, NCH, chunk, 0)
        plsc.subcore_barrier()

        @pl.when(cid == 0)
        def _():
            pltpu.sync_copy(u_s.at[pl.ds(sid * URT, URT)], u0.at[pl.ds(sid * URT, URT)])
            pltpu.sync_copy(den0_s.at[pl.ds(sid * DCH, DCH)], d00.at[pl.ds(sid * DCH, DCH)])
            pltpu.sync_copy(den1_s.at[pl.ds(sid * DCH, DCH)], d10.at[pl.ds(sid * DCH, DCH)])

        @pl.when(cid == 1)
        def _():
            pltpu.sync_copy(u_s.at[pl.ds(sid * URT, URT)], u1.at[pl.ds(sid * URT, URT)])
            pltpu.sync_copy(den0_s.at[pl.ds(sid * DCH, DCH)], d01.at[pl.ds(sid * DCH, DCH)])
            pltpu.sync_copy(den1_s.at[pl.ds(sid * DCH, DCH)], d11.at[pl.ds(sid * DCH, DCH)])

    return pl.kernel(
        body,
        out_type=(
            jax.ShapeDtypeStruct((NU, cp), jnp.float32),
            jax.ShapeDtypeStruct((NU, cp), jnp.float32),
            jax.ShapeDtypeStruct((NU,), jnp.float32),
            jax.ShapeDtypeStruct((NU,), jnp.float32),
            jax.ShapeDtypeStruct((NU,), jnp.float32),
            jax.ShapeDtypeStruct((NU,), jnp.float32),
        ),
        mesh=mesh,
        scratch_types=[
            pltpu.VMEM((CH,), jnp.int32),
            pltpu.VMEM((CH,), jnp.int32),
            pltpu.VMEM((CH,), jnp.int32),
            pltpu.VMEM((CH,), jnp.int32),
            pltpu.VMEM((CH,), jnp.int32),
            pltpu.VMEM((CH,), jnp.float32),
            pltpu.VMEM((CH,), jnp.float32),
            pltpu.VMEM((CH,), jnp.float32),
            pltpu.VMEM((CH,), jnp.float32),
            pltpu.VMEM((CH,), jnp.float32),
            pltpu.VMEM((CH,), jnp.float32),
            pltpu.VMEM((CH, cp), jnp.float32),
            pltpu.VMEM((16,), jnp.float32),
            pltpu.VMEM((16,), jnp.float32),
            pltpu.VMEM((zr, cp), jnp.float32),
            pltpu.VMEM((DCH,), jnp.float32),
            pltpu.VMEM_SHARED((NU, cp), jnp.float32),
            pltpu.VMEM_SHARED((NU,), jnp.float32),
            pltpu.VMEM_SHARED((NU,), jnp.float32),
            pltpu.VMEM_SHARED((tbl,), jnp.float32),
            pltpu.VMEM_SHARED((tbl,), jnp.float32),
            pltpu.VMEM_SHARED((tbl,), jnp.float32),
            pltpu.VMEM_SHARED((tbl,), jnp.float32),
            pltpu.SemaphoreType.DMA,
        ],
    )


RB = 400
NB = N // RB


@functools.lru_cache(maxsize=None)
def _make_post_rgat(cp):
    def body(u00, u01, u10, u11, d00, d01, d10, d11, b_ref, o_ref):
        den0 = d00[0, 0] + d01[0, 0] + 1e-16
        den1 = d10[0, 0] + d11[0, 0] + 1e-16
        num0 = u00[...] + u01[...]
        num1 = u10[...] + u11[...]
        o = 0.5 * (num0 / den0[:, None] + num1 / den1[:, None]) + b_ref[0][None, :]
        o_ref[...] = jnp.where(o > 0, o, jnp.exp(o) - 1.0)

    ub = pl.BlockSpec((RB, cp), lambda i: (i, 0))
    db = pl.BlockSpec((1, 1, RB), lambda i: (i, 0, 0))
    return pl.pallas_call(
        body,
        grid=(NB,),
        in_specs=[ub, ub, ub, ub, db, db, db, db,
                  pl.BlockSpec((1, cp), lambda i: (0, 0))],
        out_specs=ub,
        out_shape=jax.ShapeDtypeStruct((N, cp), jnp.float32),
    )


@functools.lru_cache(maxsize=None)
def _make_post_pair():
    def body(u0, u1, d00, d01, d10, d11, b_ref, o_ref):
        den0 = d00[0, 0] + d01[0, 0] + 1e-16
        den1 = d10[0, 0] + d11[0, 0] + 1e-16
        num = u0[...] + u1[...]
        o = 0.5 * (num[:, :48] / den0[:, None] + num[:, 48:96] / den1[:, None]) \
            + b_ref[0][None, :]
        o_ref[...] = jnp.where(o > 0, o, jnp.exp(o) - 1.0)

    ub = pl.BlockSpec((RB, 128), lambda i: (i, 0))
    db = pl.BlockSpec((1, 1, RB), lambda i: (i, 0, 0))
    return pl.pallas_call(
        body,
        grid=(NB,),
        in_specs=[ub, ub, db, db, db, db,
                  pl.BlockSpec((1, 48), lambda i: (0, 0))],
        out_specs=pl.BlockSpec((RB, 48), lambda i: (i, 0)),
        out_shape=jax.ShapeDtypeStruct((N, 48), jnp.float32),
    )


@functools.lru_cache(maxsize=None)
def _make_post_self(cp):
    def body(u0, u1, d0, d1, b_ref, o_ref):
        den = d0[0, 0] + d1[0, 0] + 1e-16
        num = u0[...] + u1[...]
        o = num / den[:, None] + b_ref[0][None, :]
        o_ref[...] = jnp.where(o > 0, o, jnp.exp(o) - 1.0)

    ub = pl.BlockSpec((RB, cp), lambda i: (i, 0))
    db = pl.BlockSpec((1, 1, RB), lambda i: (i, 0, 0))
    return pl.pallas_call(
        body,
        grid=(NB,),
        in_specs=[ub, ub, db, db, pl.BlockSpec((1, cp), lambda i: (0, 0))],
        out_specs=ub,
        out_shape=jax.ShapeDtypeStruct((N, cp), jnp.float32),
    )


def _leaky(z):
    return jnp.maximum(z, 0.2 * z)


def _pad_rows(a):
    return jnp.pad(a, ((0, NP - a.shape[0]), (0, 0)))


def _prep_w(w, q, k, heads, outc, cp):
    d = w.shape[1]
    wp = w.reshape(R, d, heads, outc).transpose(2, 0, 1, 3)
    qp = q.reshape(R, heads, outc).transpose(1, 0, 2)[:, :, None, :]
    kp = k.reshape(R, heads, outc).transpose(1, 0, 2)[:, :, None, :]
    if cp != outc:
        pad = ((0, 0), (0, 0), (0, 0), (0, cp - outc))
        wp = jnp.pad(wp, pad)
        qp = jnp.pad(qp, pad)
        kp = jnp.pad(kp, pad)
    return wp, qp, kp


def _rgat(xp, w, q, k, b, src, dst, typ, outc, cp):
    wp, qp, kp = _prep_w(w, q, k, HEADS, outc, cp)
    xr, qt, kt, qm, km = _make_pre(HEADS, R, cp)(xp, wp, qp, kp)
    xrh = xr.reshape(HEADS, R * NP, cp)
    qtab = qt.reshape(HEADS, R * NP)
    ktab = kt.reshape(HEADS, R * NP)
    mh = _leaky(qm.reshape(HEADS, -1).max(axis=1) + km.reshape(HEADS, -1).max(axis=1))
    edge = _make_edge(R * NP, cp, NP)
    us, ds_ = [], []
    for h in range(HEADS):
        m16 = jnp.broadcast_to(jnp.reshape(mh[h], (1,)), (16,))
        u0, u1, d0, d1 = edge(qtab[h], ktab[h], src, dst, typ, xrh[h], m16)
        us += [u0, u1]
        ds_ += [d0[:N].reshape(NB, 1, RB), d1[:N].reshape(NB, 1, RB)]
    bp = jnp.pad(b, (0, cp - b.shape[0]))[None, :]
    return _make_post_rgat(cp)(us[0], us[1], us[2], us[3],
                               ds_[0], ds_[1], ds_[2], ds_[3], bp)


def _rgat_pair(xp, w, q, k, b, src, dst, typ):
    d = w.shape[1]
    wp4 = jnp.zeros((R, d, 128), jnp.float32)
    wp4 = wp4.at[:, :, 0:OUTS].set(w.reshape(R, d, HEADS, OUTS)[:, :, 0])
    wp4 = wp4.at[:, :, 48:48 + OUTS].set(w.reshape(R, d, HEADS, OUTS)[:, :, 1])
    qp4 = jnp.zeros((HEADS, R, 128), jnp.float32)
    qp4 = qp4.at[0, :, 0:OUTS].set(q.reshape(R, HEADS, OUTS)[:, 0])
    qp4 = qp4.at[1, :, 48:48 + OUTS].set(q.reshape(R, HEADS, OUTS)[:, 1])
    kp4 = jnp.zeros((HEADS, R, 128), jnp.float32)
    kp4 = kp4.at[0, :, 0:OUTS].set(k.reshape(R, HEADS, OUTS)[:, 0])
    kp4 = kp4.at[1, :, 48:48 + OUTS].set(k.reshape(R, HEADS, OUTS)[:, 1])
    wp = jnp.broadcast_to(wp4[None], (HEADS, R, d, 128))
    xr, qt, kt, qm, km = _make_pre(HEADS, R, 128)(xp, wp, qp4[:, :, None, :], kp4[:, :, None, :])
    xrh = xr.reshape(HEADS, R * NP, 128)
    qtab = qt.reshape(HEADS, R * NP)
    ktab = kt.reshape(HEADS, R * NP)
    mh = _leaky(qm.reshape(HEADS, -1).max(axis=1) + km.reshape(HEADS, -1).max(axis=1))
    m0 = jnp.broadcast_to(jnp.reshape(mh[0], (1,)), (16,))
    m1 = jnp.broadcast_to(jnp.reshape(mh[1], (1,)), (16,))
    u0, u1, d00, d01, d10, d11 = _make_edge2(R * NP, NP)(
        qtab[0], qtab[1], ktab[0], ktab[1], src, dst, typ, xrh[0], m0, m1)
    bp = jnp.pad(b, (0, 48 - OUTS))[None, :]
    return _make_post_pair()(
        u0, u1,
        d00[:N].reshape(NB, 1, RB), d01[:N].reshape(NB, 1, RB),
        d10[:N].reshape(NB, 1, RB), d11[:N].reshape(NB, 1, RB), bp)


def kernel(x, edge_index, edge_type, W1, Q1, K1, b1, WS, QS, KS, bS,
           W2, Q2, K2, b2, WT, QT, KT, bT, Wr, a_src, a_dst, br):
    src = edge_index[0]
    dst = edge_index[1]
    typ = edge_type
    xp = _pad_rows(x)

    x_s = _rgat(xp, W1, Q1, K1, b1, src, dst, typ, HID, HID)
    x_in = _rgat_pair(_pad_rows(x_s), WS, QS, KS, bS, src, dst, typ)[:, :OUTS]

    x_t = _rgat(xp, W2, Q2, K2, b2, src, dst, typ, HID, HID)
    x_out = _rgat_pair(_pad_rows(x_t), WT, QT, KT, bT, src, dst, typ)[:, :OUTS]

    wr = Wr[None, None]
    qp = a_dst[None, None, None, :]
    kp = a_src[None, None, None, :]
    xr, qt, kt, qm, km = _make_pre(1, 1, HID)(xp, wr, qp, kp)
    m0 = _leaky(qm.max() + km.max())
    m16 = jnp.broadcast_to(jnp.reshape(m0, (1,)), (16,))
    u0, u1, d0, d1 = _make_edge(NP, HID, 0)(
        qt.reshape(NP), kt.reshape(NP), src, dst, typ, xr.reshape(NP, HID), m16)
    x_self = _make_post_self(HID)(u0, u1, d0[:N].reshape(NB, 1, RB), d1[:N].reshape(NB, 1, RB),
                                  br[None, :])
    return (x_in, x_out, x_self)


# speedup vs baseline: 3.6427x; 1.8675x over previous
import functools

import jax
import jax.numpy as jnp
from jax import lax
from jax.experimental import pallas as pl
from jax.experimental.pallas import tpu as pltpu
from jax.experimental.pallas import tpu_sc as plsc

N = 10000
E = 320000
IN = 128
HEADS = 2
R = 4
HID = 128
OUTS = 33
CPS = 128
NP = 10240
TN = 256
NT = NP // TN

NC = 2
NS = 16
NW = NC * NS
ET = E // NW
CH = 80
NCH = ET // CH
NPAIR = NCH // 2
NU = 10240
URT = NU // NS
DCH = NU // NS


@functools.lru_cache(maxsize=None)
def _make_pre(h_, r_, cp):
    def body(x_ref, w_ref, q_ref, k_ref, xr_ref, qt_ref, kt_ref, qm_ref, km_ref):
        xr = jnp.dot(x_ref[...], w_ref[0, 0], preferred_element_type=jnp.float32)
        xr_ref[0, 0] = xr
        qv = jnp.sum(xr * q_ref[0, 0, 0][None, :], axis=1)
        kv = jnp.sum(xr * k_ref[0, 0, 0][None, :], axis=1)
        qt_ref[0, 0] = qv
        kt_ref[0, 0] = kv
        qm_ref[0, 0] = jnp.full((TN,), jnp.max(qv), jnp.float32)
        km_ref[0, 0] = jnp.full((TN,), jnp.max(kv), jnp.float32)

    g = h_ * r_ * NT
    return pl.pallas_call(
        body,
        grid=(h_, r_, NT),
        in_specs=[
            pl.BlockSpec((TN, IN), lambda h, r, i: (i, 0)),
            pl.BlockSpec((1, 1, IN, cp), lambda h, r, i: (h, r, 0, 0)),
            pl.BlockSpec((1, 1, 1, cp), lambda h, r, i: (h, r, 0, 0)),
            pl.BlockSpec((1, 1, 1, cp), lambda h, r, i: (h, r, 0, 0)),
        ],
        out_specs=[
            pl.BlockSpec((1, 1, TN, cp), lambda h, r, i: (h, r, i, 0)),
            pl.BlockSpec((1, 1, TN), lambda h, r, i: ((h * r_ + r) * NT + i, 0, 0)),
            pl.BlockSpec((1, 1, TN), lambda h, r, i: ((h * r_ + r) * NT + i, 0, 0)),
            pl.BlockSpec((1, 1, TN), lambda h, r, i: ((h * r_ + r) * NT + i, 0, 0)),
            pl.BlockSpec((1, 1, TN), lambda h, r, i: ((h * r_ + r) * NT + i, 0, 0)),
        ],
        out_shape=[
            jax.ShapeDtypeStruct((h_, r_, NP, cp), jnp.float32),
            jax.ShapeDtypeStruct((g, 1, TN), jnp.float32),
            jax.ShapeDtypeStruct((g, 1, TN), jnp.float32),
            jax.ShapeDtypeStruct((g, 1, TN), jnp.float32),
            jax.ShapeDtypeStruct((g, 1, TN), jnp.float32),
        ],
    )


@functools.lru_cache(maxsize=None)
def _make_edge(tbl, cp, tstride):
    mesh = plsc.VectorSubcoreMesh(core_axis_name="c", subcore_axis_name="s")
    stg = tbl // NS
    zr = 64
    nvec = cp // 16

    def body(qtab_h, ktab_h, src_h, dst_h, typ_h, xr_h, m_h,
             u0, u1, d0, d1,
             src_v, dst_v, typ_v, gsrc_v, gdst_v, dsc_v, q_v, k_v, ex_v, rows_v,
             m_v, zb_v, zd_v, u_s, den_s, qtab_s, ktab_s,
             isem, rsem, qsem, ksem, dsem, ssem):
        cid = lax.axis_index("c")
        sid = lax.axis_index("s")
        wid = cid * NS + sid

        pltpu.sync_copy(qtab_h.at[pl.ds(sid * stg, stg)], qtab_s.at[pl.ds(sid * stg, stg)])
        pltpu.sync_copy(ktab_h.at[pl.ds(sid * stg, stg)], ktab_s.at[pl.ds(sid * stg, stg)])
        pltpu.sync_copy(m_h, m_v)

        def zrow(i, c):
            for j in range(nvec):
                zb_v[i, pl.ds(j * 16, 16)] = jnp.zeros((16,), jnp.float32)
            return c
        lax.fori_loop(0, zr, zrow, 0)

        def zden(i, c):
            zd_v[pl.ds(i * 16, 16)] = jnp.zeros((16,), jnp.float32)
            return c
        lax.fori_loop(0, DCH // 16, zden, 0)

        for j in range(URT // zr):
            pltpu.sync_copy(zb_v, u_s.at[pl.ds(sid * URT + j * zr, zr)])
        pltpu.sync_copy(zd_v, den_s.at[pl.ds(sid * DCH, DCH)])

        plsc.subcore_barrier()
        mv = m_v[...]

        def load_idx(c, sl4):
            base = wid * ET + c * CH
            pltpu.async_copy(src_h.at[pl.ds(base, CH)], src_v.at[sl4], isem.at[sl4])
            pltpu.async_copy(dst_h.at[pl.ds(base, CH)], dst_v.at[sl4], isem.at[sl4])
            pltpu.async_copy(typ_h.at[pl.ds(base, CH)], typ_v.at[sl4], isem.at[sl4])

        def wait_idx(c, sl4):
            base = wid * ET + c * CH
            pltpu.make_async_copy(src_h.at[pl.ds(base, CH)], src_v.at[sl4], isem.at[sl4]).wait()
            pltpu.make_async_copy(dst_h.at[pl.ds(base, CH)], dst_v.at[sl4], isem.at[sl4]).wait()
            pltpu.make_async_copy(typ_h.at[pl.ds(base, CH)], typ_v.at[sl4], isem.at[sl4]).wait()

        def gidx(sl4, s):
            for i in range(CH // 16):
                sl = pl.ds(i * 16, 16)
                tv = typ_v[sl4, sl]
                gsrc_v[s, sl] = tv * tstride + src_v[sl4, sl]
                gdst_v[s, sl] = tv * tstride + dst_v[sl4, sl]
                dsc_v[s, sl] = dst_v[sl4, sl]

        def fire_gathers(s):
            return (pltpu.async_copy(xr_h.at[gsrc_v.at[s]], rows_v.at[s], rsem.at[s]),
                    pltpu.async_copy(qtab_s.at[gdst_v.at[s]], q_v.at[s], qsem.at[s]),
                    pltpu.async_copy(ktab_s.at[gsrc_v.at[s]], k_v.at[s], ksem.at[s]))

        def compute_ex(s):
            for i in range(CH // 16):
                sl = pl.ds(i * 16, 16)
                z = q_v[s, sl] + k_v[s, sl]
                z = jnp.maximum(z, 0.2 * z)
                ex_v[s, sl] = jnp.exp(z - mv)

        def scale(s):
            def sbody(g, c2):
                exg = ex_v[s, pl.ds(g * 16, 16)]
                for i in range(16):
                    sc = exg[i]
                    for j in range(nvec):
                        slj = pl.ds(j * 16, 16)
                        rows_v[s, g * 16 + i, slj] = rows_v[s, g * 16 + i, slj] * sc
                return c2
            lax.fori_loop(0, CH // 16, sbody, 0)

        def pair(g, carry):
            sA = lax.rem(2 * g, 4)

            @pl.when(g < NPAIR - 1)
            def _():
                load_idx(2 * g + 2, lax.rem(sA + 2, 4))
                load_idx(2 * g + 3, lax.rem(sA + 3, 4))
            wait_idx(2 * g, sA)
            gidx(sA, 0)
            gA = fire_gathers(0)
            wait_idx(2 * g + 1, sA + 1)
            gidx(sA + 1, 1)
            gB = fire_gathers(1)
            gA[1].wait()
            gA[2].wait()
            compute_ex(0)
            dA = pltpu.async_copy(ex_v.at[0], den_s.at[dsc_v.at[0]], dsem.at[0], add=True)
            gA[0].wait()
            scale(0)
            sA = pltpu.async_copy(rows_v.at[0], u_s.at[dsc_v.at[0]], ssem.at[0], add=True)
            gB[1].wait()
            gB[2].wait()
            compute_ex(1)
            dB = pltpu.async_copy(ex_v.at[1], den_s.at[dsc_v.at[1]], dsem.at[1], add=True)
            gB[0].wait()
            scale(1)
            sB = pltpu.async_copy(rows_v.at[1], u_s.at[dsc_v.at[1]], ssem.at[1], add=True)
            dA.wait()
            sA.wait()
            dB.wait()
            sB.wait()
            return carry

        load_idx(0, 0)
        load_idx(1, 1)
        lax.fori_loop(0, NPAIR, pair, 0)
        if NCH % 2 == 1:
            load_idx(NCH - 1, 0)
            wait_idx(NCH - 1, 0)
            gidx(0, 0)
            gA = fire_gathers(0)
            gA[1].wait()
            gA[2].wait()
            compute_ex(0)
            pltpu.sync_copy(ex_v.at[0], den_s.at[dsc_v.at[0]], add=True)
            gA[0].wait()
            scale(0)
            pltpu.sync_copy(rows_v.at[0], u_s.at[dsc_v.at[0]], add=True)
        plsc.subcore_barrier()

        @pl.when(cid == 0)
        def _():
            pltpu.sync_copy(u_s.at[pl.ds(sid * URT, URT)], u0.at[pl.ds(sid * URT, URT)])
            pltpu.sync_copy(den_s.at[pl.ds(sid * DCH, DCH)], d0.at[pl.ds(sid * DCH, DCH)])

        @pl.when(cid == 1)
        def _():
            pltpu.sync_copy(u_s.at[pl.ds(sid * URT, URT)], u1.at[pl.ds(sid * URT, URT)])
            pltpu.sync_copy(den_s.at[pl.ds(sid * DCH, DCH)], d1.at[pl.ds(sid * DCH, DCH)])

    return pl.kernel(
        body,
        out_type=(
            jax.ShapeDtypeStruct((NU, cp), jnp.float32),
            jax.ShapeDtypeStruct((NU, cp), jnp.float32),
            jax.ShapeDtypeStruct((NU,), jnp.float32),
            jax.ShapeDtypeStruct((NU,), jnp.float32),
        ),
        mesh=mesh,
        scratch_types=[
            pltpu.VMEM((4, CH), jnp.int32),
            pltpu.VMEM((4, CH), jnp.int32),
            pltpu.VMEM((4, CH), jnp.int32),
            pltpu.VMEM((2, CH), jnp.int32),
            pltpu.VMEM((2, CH), jnp.int32),
            pltpu.VMEM((2, CH), jnp.int32),
            pltpu.VMEM((2, CH), jnp.float32),
            pltpu.VMEM((2, CH), jnp.float32),
            pltpu.VMEM((2, CH), jnp.float32),
            pltpu.VMEM((2, CH, cp), jnp.float32),
            pltpu.VMEM((16,), jnp.float32),
            pltpu.VMEM((zr, cp), jnp.float32),
            pltpu.VMEM((DCH,), jnp.float32),
            pltpu.VMEM_SHARED((NU, cp), jnp.float32),
            pltpu.VMEM_SHARED((NU,), jnp.float32),
            pltpu.VMEM_SHARED((tbl,), jnp.float32),
            pltpu.VMEM_SHARED((tbl,), jnp.float32),
            pltpu.SemaphoreType.DMA((4,)),
            pltpu.SemaphoreType.DMA((2,)),
            pltpu.SemaphoreType.DMA((2,)),
            pltpu.SemaphoreType.DMA((2,)),
            pltpu.SemaphoreType.DMA((2,)),
            pltpu.SemaphoreType.DMA((2,)),
        ],
    )


@functools.lru_cache(maxsize=None)
def _make_edge2(tbl, tstride):
    cp = 128
    mesh = plsc.VectorSubcoreMesh(core_axis_name="c", subcore_axis_name="s")
    stg = tbl // NS
    zr = 64
    nvec = cp // 16

    def body(qt0_h, qt1_h, kt0_h, kt1_h, src_h, dst_h, typ_h, xr_h, m0_h, m1_h,
             u0, u1, d00, d01, d10, d11,
             src_v, dst_v, typ_v, gsrc_v, gdst_v, dsc_v, q0_v, q1_v, k0_v, k1_v,
             ex0_v, ex1_v, rows_v, m0_v, m1_v, zb_v, zd_v,
             u_s, den0_s, den1_s, qt0_s, qt1_s, kt0_s, kt1_s,
             isem, rsem, q0sem, q1sem, k0sem, k1sem, d0sem, d1sem, ssem):
        cid = lax.axis_index("c")
        sid = lax.axis_index("s")
        wid = cid * NS + sid

        sls = pl.ds(sid * stg, stg)
        pltpu.sync_copy(qt0_h.at[sls], qt0_s.at[sls])
        pltpu.sync_copy(qt1_h.at[sls], qt1_s.at[sls])
        pltpu.sync_copy(kt0_h.at[sls], kt0_s.at[sls])
        pltpu.sync_copy(kt1_h.at[sls], kt1_s.at[sls])
        pltpu.sync_copy(m0_h, m0_v)
        pltpu.sync_copy(m1_h, m1_v)

        def zrow(i, c):
            for j in range(nvec):
                zb_v[i, pl.ds(j * 16, 16)] = jnp.zeros((16,), jnp.float32)
            return c
        lax.fori_loop(0, zr, zrow, 0)

        def zden(i, c):
            zd_v[pl.ds(i * 16, 16)] = jnp.zeros((16,), jnp.float32)
            return c
        lax.fori_loop(0, DCH // 16, zden, 0)

        for j in range(URT // zr):
            pltpu.sync_copy(zb_v, u_s.at[pl.ds(sid * URT + j * zr, zr)])
        pltpu.sync_copy(zd_v, den0_s.at[pl.ds(sid * DCH, DCH)])
        pltpu.sync_copy(zd_v, den1_s.at[pl.ds(sid * DCH, DCH)])

        plsc.subcore_barrier()
        mv0 = m0_v[...]
        mv1 = m1_v[...]

        def load_idx(c, sl4):
            base = wid * ET + c * CH
            pltpu.async_copy(src_h.at[pl.ds(base, CH)], src_v.at[sl4], isem.at[sl4])
            pltpu.async_copy(dst_h.at[pl.ds(base, CH)], dst_v.at[sl4], isem.at[sl4])
            pltpu.async_copy(typ_h.at[pl.ds(base, CH)], typ_v.at[sl4], isem.at[sl4])

        def wait_idx(c, sl4):
            base = wid * ET + c * CH
            pltpu.make_async_copy(src_h.at[pl.ds(base, CH)], src_v.at[sl4], isem.at[sl4]).wait()
            pltpu.make_async_copy(dst_h.at[pl.ds(base, CH)], dst_v.at[sl4], isem.at[sl4]).wait()
            pltpu.make_async_copy(typ_h.at[pl.ds(base, CH)], typ_v.at[sl4], isem.at[sl4]).wait()

        def gidx(sl4, s):
            for i in range(CH // 16):
                sl = pl.ds(i * 16, 16)
                tv = typ_v[sl4, sl]
                gsrc_v[s, sl] = tv * tstride + src_v[sl4, sl]
                gdst_v[s, sl] = tv * tstride + dst_v[sl4, sl]
                dsc_v[s, sl] = dst_v[sl4, sl]

        def fire_gathers(s):
            return (pltpu.async_copy(xr_h.at[gsrc_v.at[s]], rows_v.at[s], rsem.at[s]),
                    pltpu.async_copy(qt0_s.at[gdst_v.at[s]], q0_v.at[s], q0sem.at[s]),
                    pltpu.async_copy(qt1_s.at[gdst_v.at[s]], q1_v.at[s], q1sem.at[s]),
                    pltpu.async_copy(kt0_s.at[gsrc_v.at[s]], k0_v.at[s], k0sem.at[s]),
                    pltpu.async_copy(kt1_s.at[gsrc_v.at[s]], k1_v.at[s], k1sem.at[s]))

        def compute_ex(s):
            for i in range(CH // 16):
                sl = pl.ds(i * 16, 16)
                z0 = q0_v[s, sl] + k0_v[s, sl]
                z0 = jnp.maximum(z0, 0.2 * z0)
                ex0_v[s, sl] = jnp.exp(z0 - mv0)
                z1 = q1_v[s, sl] + k1_v[s, sl]
                z1 = jnp.maximum(z1, 0.2 * z1)
                ex1_v[s, sl] = jnp.exp(z1 - mv1)

        def scale(s):
            def sbody(g, c2):
                exg0 = ex0_v[s, pl.ds(g * 16, 16)]
                exg1 = ex1_v[s, pl.ds(g * 16, 16)]
                for i in range(16):
                    s0 = exg0[i]
                    s1 = exg1[i]
                    for j in range(3):
                        slj = pl.ds(j * 16, 16)
                        rows_v[s, g * 16 + i, slj] = rows_v[s, g * 16 + i, slj] * s0
                    for j in range(3, 6):
                        slj = pl.ds(j * 16, 16)
                        rows_v[s, g * 16 + i, slj] = rows_v[s, g * 16 + i, slj] * s1
                return c2
            lax.fori_loop(0, CH // 16, sbody, 0)

        def pair(g, carry):
            sA = lax.rem(2 * g, 4)

            @pl.when(g < NPAIR - 1)
            def _():
                load_idx(2 * g + 2, lax.rem(sA + 2, 4))
                load_idx(2 * g + 3, lax.rem(sA + 3, 4))
            wait_idx(2 * g, sA)
            gidx(sA, 0)
            gA = fire_gathers(0)
            wait_idx(2 * g + 1, sA + 1)
            gidx(sA + 1, 1)
            gB = fire_gathers(1)
            for w in gA[1:]:
                w.wait()
            compute_ex(0)
            dA0 = pltpu.async_copy(ex0_v.at[0], den0_s.at[dsc_v.at[0]], d0sem.at[0], add=True)
            dA1 = pltpu.async_copy(ex1_v.at[0], den1_s.at[dsc_v.at[0]], d1sem.at[0], add=True)
            gA[0].wait()
            scale(0)
            sA = pltpu.async_copy(rows_v.at[0], u_s.at[dsc_v.at[0]], ssem.at[0], add=True)
            for w in gB[1:]:
                w.wait()
            compute_ex(1)
            dB0 = pltpu.async_copy(ex0_v.at[1], den0_s.at[dsc_v.at[1]], d0sem.at[1], add=True)
            dB1 = pltpu.async_copy(ex1_v.at[1], den1_s.at[dsc_v.at[1]], d1sem.at[1], add=True)
            gB[0].wait()
            scale(1)
            sB = pltpu.async_copy(rows_v.at[1], u_s.at[dsc_v.at[1]], ssem.at[1], add=True)
            dA0.wait()
            dA1.wait()
            sA.wait()
            dB0.wait()
            dB1.wait()
            sB.wait()
            return carry

        load_idx(0, 0)
        load_idx(1, 1)
        lax.fori_loop(0, NPAIR, pair, 0)
        if NCH % 2 == 1:
            load_idx(NCH - 1, 0)
            wait_idx(NCH - 1, 0)
            gidx(0, 0)
            gA = fire_gathers(0)
            for w in gA[1:]:
                w.wait()
            compute_ex(0)
            pltpu.sync_copy(ex0_v.at[0], den0_s.at[dsc_v.at[0]], add=True)
            pltpu.sync_copy(ex1_v.at[0], den1_s.at[dsc_v.at[0]], add=True)
            gA[0].wait()
            scale(0)
            pltpu.sync_copy(rows_v.at[0], u_s.at[dsc_v.at[0]], add=True)
        plsc.subcore_barrier()

        @pl.when(cid == 0)
        def _():
            pltpu.sync_copy(u_s.at[pl.ds(sid * URT, URT)], u0.at[pl.ds(sid * URT, URT)])
            pltpu.sync_copy(den0_s.at[pl.ds(sid * DCH, DCH)], d00.at[pl.ds(sid * DCH, DCH)])
            pltpu.sync_copy(den1_s.at[pl.ds(sid * DCH, DCH)], d10.at[pl.ds(sid * DCH, DCH)])

        @pl.when(cid == 1)
        def _():
            pltpu.sync_copy(u_s.at[pl.ds(sid * URT, URT)], u1.at[pl.ds(sid * URT, URT)])
            pltpu.sync_copy(den0_s.at[pl.ds(sid * DCH, DCH)], d01.at[pl.ds(sid * DCH, DCH)])
            pltpu.sync_copy(den1_s.at[pl.ds(sid * DCH, DCH)], d11.at[pl.ds(sid * DCH, DCH)])

    return pl.kernel(
        body,
        out_type=(
            jax.ShapeDtypeStruct((NU, cp), jnp.float32),
            jax.ShapeDtypeStruct((NU, cp), jnp.float32),
            jax.ShapeDtypeStruct((NU,), jnp.float32),
            jax.ShapeDtypeStruct((NU,), jnp.float32),
            jax.ShapeDtypeStruct((NU,), jnp.float32),
            jax.ShapeDtypeStruct((NU,), jnp.float32),
        ),
        mesh=mesh,
        scratch_types=[
            pltpu.VMEM((4, CH), jnp.int32),
            pltpu.VMEM((4, CH), jnp.int32),
            pltpu.VMEM((4, CH), jnp.int32),
            pltpu.VMEM((2, CH), jnp.int32),
            pltpu.VMEM((2, CH), jnp.int32),
            pltpu.VMEM((2, CH), jnp.int32),
            pltpu.VMEM((2, CH), jnp.float32),
            pltpu.VMEM((2, CH), jnp.float32),
            pltpu.VMEM((2, CH), jnp.float32),
            pltpu.VMEM((2, CH), jnp.float32),
            pltpu.VMEM((2, CH), jnp.float32),
            pltpu.VMEM((2, CH), jnp.float32),
            pltpu.VMEM((2, CH, cp), jnp.float32),
            pltpu.VMEM((16,), jnp.float32),
            pltpu.VMEM((16,), jnp.float32),
            pltpu.VMEM((zr, cp), jnp.float32),
            pltpu.VMEM((DCH,), jnp.float32),
            pltpu.VMEM_SHARED((NU, cp), jnp.float32),
            pltpu.VMEM_SHARED((NU,), jnp.float32),
            pltpu.VMEM_SHARED((NU,), jnp.float32),
            pltpu.VMEM_SHARED((tbl,), jnp.float32),
            pltpu.VMEM_SHARED((tbl,), jnp.float32),
            pltpu.VMEM_SHARED((tbl,), jnp.float32),
            pltpu.VMEM_SHARED((tbl,), jnp.float32),
            pltpu.SemaphoreType.DMA((4,)),
            pltpu.SemaphoreType.DMA((2,)),
            pltpu.SemaphoreType.DMA((2,)),
            pltpu.SemaphoreType.DMA((2,)),
            pltpu.SemaphoreType.DMA((2,)),
            pltpu.SemaphoreType.DMA((2,)),
            pltpu.SemaphoreType.DMA((2,)),
            pltpu.SemaphoreType.DMA((2,)),
            pltpu.SemaphoreType.DMA((2,)),
        ],
    )


RB = 400
NB = N // RB


@functools.lru_cache(maxsize=None)
def _make_post_rgat(cp):
    def body(u00, u01, u10, u11, d00, d01, d10, d11, b_ref, o_ref):
        den0 = d00[0, 0] + d01[0, 0] + 1e-16
        den1 = d10[0, 0] + d11[0, 0] + 1e-16
        num0 = u00[...] + u01[...]
        num1 = u10[...] + u11[...]
        o = 0.5 * (num0 / den0[:, None] + num1 / den1[:, None]) + b_ref[0][None, :]
        o_ref[...] = jnp.where(o > 0, o, jnp.exp(o) - 1.0)

    ub = pl.BlockSpec((RB, cp), lambda i: (i, 0))
    db = pl.BlockSpec((1, 1, RB), lambda i: (i, 0, 0))
    return pl.pallas_call(
        body,
        grid=(NB,),
        in_specs=[ub, ub, ub, ub, db, db, db, db,
                  pl.BlockSpec((1, cp), lambda i: (0, 0))],
        out_specs=ub,
        out_shape=jax.ShapeDtypeStruct((N, cp), jnp.float32),
    )


@functools.lru_cache(maxsize=None)
def _make_post_pair():
    def body(u0, u1, d00, d01, d10, d11, b_ref, o_ref):
        den0 = d00[0, 0] + d01[0, 0] + 1e-16
        den1 = d10[0, 0] + d11[0, 0] + 1e-16
        num = u0[...] + u1[...]
        o = 0.5 * (num[:, :48] / den0[:, None] + num[:, 48:96] / den1[:, None]) \
            + b_ref[0][None, :]
        o_ref[...] = jnp.where(o > 0, o, jnp.exp(o) - 1.0)

    ub = pl.BlockSpec((RB, 128), lambda i: (i, 0))
    db = pl.BlockSpec((1, 1, RB), lambda i: (i, 0, 0))
    return pl.pallas_call(
        body,
        grid=(NB,),
        in_specs=[ub, ub, db, db, db, db,
                  pl.BlockSpec((1, 48), lambda i: (0, 0))],
        out_specs=pl.BlockSpec((RB, 48), lambda i: (i, 0)),
        out_shape=jax.ShapeDtypeStruct((N, 48), jnp.float32),
    )


@functools.lru_cache(maxsize=None)
def _make_post_self(cp):
    def body(u0, u1, d0, d1, b_ref, o_ref):
        den = d0[0, 0] + d1[0, 0] + 1e-16
        num = u0[...] + u1[...]
        o = num / den[:, None] + b_ref[0][None, :]
        o_ref[...] = jnp.where(o > 0, o, jnp.exp(o) - 1.0)

    ub = pl.BlockSpec((RB, cp), lambda i: (i, 0))
    db = pl.BlockSpec((1, 1, RB), lambda i: (i, 0, 0))
    return pl.pallas_call(
        body,
        grid=(NB,),
        in_specs=[ub, ub, db, db, pl.BlockSpec((1, cp), lambda i: (0, 0))],
        out_specs=ub,
        out_shape=jax.ShapeDtypeStruct((N, cp), jnp.float32),
    )


def _leaky(z):
    return jnp.maximum(z, 0.2 * z)


def _pad_rows(a):
    return jnp.pad(a, ((0, NP - a.shape[0]), (0, 0)))


def _prep_w(w, q, k, heads, outc, cp):
    d = w.shape[1]
    wp = w.reshape(R, d, heads, outc).transpose(2, 0, 1, 3)
    qp = q.reshape(R, heads, outc).transpose(1, 0, 2)[:, :, None, :]
    kp = k.reshape(R, heads, outc).transpose(1, 0, 2)[:, :, None, :]
    if cp != outc:
        pad = ((0, 0), (0, 0), (0, 0), (0, cp - outc))
        wp = jnp.pad(wp, pad)
        qp = jnp.pad(qp, pad)
        kp = jnp.pad(kp, pad)
    return wp, qp, kp


def _rgat(xp, w, q, k, b, src, dst, typ, outc, cp):
    wp, qp, kp = _prep_w(w, q, k, HEADS, outc, cp)
    xr, qt, kt, qm, km = _make_pre(HEADS, R, cp)(xp, wp, qp, kp)
    xrh = xr.reshape(HEADS, R * NP, cp)
    qtab = qt.reshape(HEADS, R * NP)
    ktab = kt.reshape(HEADS, R * NP)
    mh = _leaky(qm.reshape(HEADS, -1).max(axis=1) + km.reshape(HEADS, -1).max(axis=1))
    edge = _make_edge(R * NP, cp, NP)
    us, ds_ = [], []
    for h in range(HEADS):
        m16 = jnp.broadcast_to(jnp.reshape(mh[h], (1,)), (16,))
        u0, u1, d0, d1 = edge(qtab[h], ktab[h], src, dst, typ, xrh[h], m16)
        us += [u0, u1]
        ds_ += [d0[:N].reshape(NB, 1, RB), d1[:N].reshape(NB, 1, RB)]
    bp = jnp.pad(b, (0, cp - b.shape[0]))[None, :]
    return _make_post_rgat(cp)(us[0], us[1], us[2], us[3],
                               ds_[0], ds_[1], ds_[2], ds_[3], bp)


def _rgat_pair(xp, w, q, k, b, src, dst, typ):
    d = w.shape[1]
    wp4 = jnp.zeros((R, d, 128), jnp.float32)
    wp4 = wp4.at[:, :, 0:OUTS].set(w.reshape(R, d, HEADS, OUTS)[:, :, 0])
    wp4 = wp4.at[:, :, 48:48 + OUTS].set(w.reshape(R, d, HEADS, OUTS)[:, :, 1])
    qp4 = jnp.zeros((HEADS, R, 128), jnp.float32)
    qp4 = qp4.at[0, :, 0:OUTS].set(q.reshape(R, HEADS, OUTS)[:, 0])
    qp4 = qp4.at[1, :, 48:48 + OUTS].set(q.reshape(R, HEADS, OUTS)[:, 1])
    kp4 = jnp.zeros((HEADS, R, 128), jnp.float32)
    kp4 = kp4.at[0, :, 0:OUTS].set(k.reshape(R, HEADS, OUTS)[:, 0])
    kp4 = kp4.at[1, :, 48:48 + OUTS].set(k.reshape(R, HEADS, OUTS)[:, 1])
    wp = jnp.broadcast_to(wp4[None], (HEADS, R, d, 128))
    xr, qt, kt, qm, km = _make_pre(HEADS, R, 128)(xp, wp, qp4[:, :, None, :], kp4[:, :, None, :])
    xrh = xr.reshape(HEADS, R * NP, 128)
    qtab = qt.reshape(HEADS, R * NP)
    ktab = kt.reshape(HEADS, R * NP)
    mh = _leaky(qm.reshape(HEADS, -1).max(axis=1) + km.reshape(HEADS, -1).max(axis=1))
    m0 = jnp.broadcast_to(jnp.reshape(mh[0], (1,)), (16,))
    m1 = jnp.broadcast_to(jnp.reshape(mh[1], (1,)), (16,))
    u0, u1, d00, d01, d10, d11 = _make_edge2(R * NP, NP)(
        qtab[0], qtab[1], ktab[0], ktab[1], src, dst, typ, xrh[0], m0, m1)
    bp = jnp.pad(b, (0, 48 - OUTS))[None, :]
    return _make_post_pair()(
        u0, u1,
        d00[:N].reshape(NB, 1, RB), d01[:N].reshape(NB, 1, RB),
        d10[:N].reshape(NB, 1, RB), d11[:N].reshape(NB, 1, RB), bp)


def kernel(x, edge_index, edge_type, W1, Q1, K1, b1, WS, QS, KS, bS,
           W2, Q2, K2, b2, WT, QT, KT, bT, Wr, a_src, a_dst, br):
    src = edge_index[0]
    dst = edge_index[1]
    typ = edge_type
    xp = _pad_rows(x)

    x_s = _rgat(xp, W1, Q1, K1, b1, src, dst, typ, HID, HID)
    x_in = _rgat_pair(_pad_rows(x_s), WS, QS, KS, bS, src, dst, typ)[:, :OUTS]

    x_t = _rgat(xp, W2, Q2, K2, b2, src, dst, typ, HID, HID)
    x_out = _rgat_pair(_pad_rows(x_t), WT, QT, KT, bT, src, dst, typ)[:, :OUTS]

    wr = Wr[None, None]
    qp = a_dst[None, None, None, :]
    kp = a_src[None, None, None, :]
    xr, qt, kt, qm, km = _make_pre(1, 1, HID)(xp, wr, qp, kp)
    m0 = _leaky(qm.max() + km.max())
    m16 = jnp.broadcast_to(jnp.reshape(m0, (1,)), (16,))
    u0, u1, d0, d1 = _make_edge(NP, HID, 0)(
        qt.reshape(NP), kt.reshape(NP), src, dst, typ, xr.reshape(NP, HID), m16)
    x_self = _make_post_self(HID)(u0, u1, d0[:N].reshape(NB, 1, RB), d1[:N].reshape(NB, 1, RB),
                                  br[None, :])
    return (x_in, x_out, x_self)


# speedup vs baseline: 3.7252x; 1.0226x over previous
import functools

import jax
import jax.numpy as jnp
from jax import lax
from jax.experimental import pallas as pl
from jax.experimental.pallas import tpu as pltpu
from jax.experimental.pallas import tpu_sc as plsc

N = 10000
E = 320000
IN = 128
HEADS = 2
R = 4
HID = 128
OUTS = 33
CPS = 128
NP = 10240
TN = 256
NT = NP // TN

NC = 2
NS = 16
NW = NC * NS
ET = E // NW
CH = 80
NCH = ET // CH
NPAIR = NCH // 2
ET2 = E // NS
NCH2 = ET2 // CH
NPAIR2 = NCH2 // 2
NU = 10240
URT = NU // NS
DCH = NU // NS


@functools.lru_cache(maxsize=None)
def _make_pre(h_, r_, cp):
    def body(x_ref, w_ref, q_ref, k_ref, xr_ref, qt_ref, kt_ref, qm_ref, km_ref):
        xr = jnp.dot(x_ref[...], w_ref[0, 0], preferred_element_type=jnp.float32)
        xr_ref[0, 0] = xr
        qv = jnp.sum(xr * q_ref[0, 0, 0][None, :], axis=1)
        kv = jnp.sum(xr * k_ref[0, 0, 0][None, :], axis=1)
        qt_ref[0, 0] = qv
        kt_ref[0, 0] = kv
        qm_ref[0, 0] = jnp.full((TN,), jnp.max(qv), jnp.float32)
        km_ref[0, 0] = jnp.full((TN,), jnp.max(kv), jnp.float32)

    g = h_ * r_ * NT
    return pl.pallas_call(
        body,
        grid=(h_, r_, NT),
        in_specs=[
            pl.BlockSpec((TN, IN), lambda h, r, i: (i, 0)),
            pl.BlockSpec((1, 1, IN, cp), lambda h, r, i: (h, r, 0, 0)),
            pl.BlockSpec((1, 1, 1, cp), lambda h, r, i: (h, r, 0, 0)),
            pl.BlockSpec((1, 1, 1, cp), lambda h, r, i: (h, r, 0, 0)),
        ],
        out_specs=[
            pl.BlockSpec((1, 1, TN, cp), lambda h, r, i: (h, r, i, 0)),
            pl.BlockSpec((1, 1, TN), lambda h, r, i: ((h * r_ + r) * NT + i, 0, 0)),
            pl.BlockSpec((1, 1, TN), lambda h, r, i: ((h * r_ + r) * NT + i, 0, 0)),
            pl.BlockSpec((1, 1, TN), lambda h, r, i: ((h * r_ + r) * NT + i, 0, 0)),
            pl.BlockSpec((1, 1, TN), lambda h, r, i: ((h * r_ + r) * NT + i, 0, 0)),
        ],
        out_shape=[
            jax.ShapeDtypeStruct((h_, r_, NP, cp), jnp.float32),
            jax.ShapeDtypeStruct((g, 1, TN), jnp.float32),
            jax.ShapeDtypeStruct((g, 1, TN), jnp.float32),
            jax.ShapeDtypeStruct((g, 1, TN), jnp.float32),
            jax.ShapeDtypeStruct((g, 1, TN), jnp.float32),
        ],
    )


@functools.lru_cache(maxsize=None)
def _make_edge(tbl, cp, tstride):
    mesh = plsc.VectorSubcoreMesh(core_axis_name="c", subcore_axis_name="s")
    stg = tbl // NS
    zr = 64
    nvec = cp // 16

    def body(qtab_h, ktab_h, src_h, dst_h, typ_h, xr_h, m_h,
             u0, u1, d0, d1,
             src_v, dst_v, typ_v, gsrc_v, gdst_v, dsc_v, q_v, k_v, ex_v, rows_v,
             m_v, zb_v, zd_v, u_s, den_s, qtab_s, ktab_s,
             isem, rsem, qsem, ksem, dsem, ssem):
        cid = lax.axis_index("c")
        sid = lax.axis_index("s")
        wid = cid * NS + sid

        pltpu.sync_copy(qtab_h.at[pl.ds(sid * stg, stg)], qtab_s.at[pl.ds(sid * stg, stg)])
        pltpu.sync_copy(ktab_h.at[pl.ds(sid * stg, stg)], ktab_s.at[pl.ds(sid * stg, stg)])
        pltpu.sync_copy(m_h, m_v)

        def zrow(i, c):
            for j in range(nvec):
                zb_v[i, pl.ds(j * 16, 16)] = jnp.zeros((16,), jnp.float32)
            return c
        lax.fori_loop(0, zr, zrow, 0)

        def zden(i, c):
            zd_v[pl.ds(i * 16, 16)] = jnp.zeros((16,), jnp.float32)
            return c
        lax.fori_loop(0, DCH // 16, zden, 0)

        for j in range(URT // zr):
            pltpu.sync_copy(zb_v, u_s.at[pl.ds(sid * URT + j * zr, zr)])
        pltpu.sync_copy(zd_v, den_s.at[pl.ds(sid * DCH, DCH)])

        plsc.subcore_barrier()
        mv = m_v[...]

        def load_idx(c, sl4):
            base = wid * ET + c * CH
            pltpu.async_copy(src_h.at[pl.ds(base, CH)], src_v.at[sl4], isem.at[sl4])
            pltpu.async_copy(dst_h.at[pl.ds(base, CH)], dst_v.at[sl4], isem.at[sl4])
            pltpu.async_copy(typ_h.at[pl.ds(base, CH)], typ_v.at[sl4], isem.at[sl4])

        def wait_idx(c, sl4):
            base = wid * ET + c * CH
            pltpu.make_async_copy(src_h.at[pl.ds(base, CH)], src_v.at[sl4], isem.at[sl4]).wait()
            pltpu.make_async_copy(dst_h.at[pl.ds(base, CH)], dst_v.at[sl4], isem.at[sl4]).wait()
            pltpu.make_async_copy(typ_h.at[pl.ds(base, CH)], typ_v.at[sl4], isem.at[sl4]).wait()

        def gidx(sl4, s):
            for i in range(CH // 16):
                sl = pl.ds(i * 16, 16)
                tv = typ_v[sl4, sl]
                gsrc_v[s, sl] = tv * tstride + src_v[sl4, sl]
                gdst_v[s, sl] = tv * tstride + dst_v[sl4, sl]
                dsc_v[s, sl] = dst_v[sl4, sl]

        def fire_gathers(s):
            return (pltpu.async_copy(xr_h.at[gsrc_v.at[s]], rows_v.at[s], rsem.at[s]),
                    pltpu.async_copy(qtab_s.at[gdst_v.at[s]], q_v.at[s], qsem.at[s]),
                    pltpu.async_copy(ktab_s.at[gsrc_v.at[s]], k_v.at[s], ksem.at[s]))

        def compute_ex(s):
            for i in range(CH // 16):
                sl = pl.ds(i * 16, 16)
                z = q_v[s, sl] + k_v[s, sl]
                z = jnp.maximum(z, 0.2 * z)
                ex_v[s, sl] = jnp.exp(z - mv)

        def scale(s):
            def sbody(g, c2):
                exg = ex_v[s, pl.ds(g * 16, 16)]
                for i in range(16):
                    sc = exg[i]
                    for j in range(nvec):
                        slj = pl.ds(j * 16, 16)
                        rows_v[s, g * 16 + i, slj] = rows_v[s, g * 16 + i, slj] * sc
                return c2
            lax.fori_loop(0, CH // 16, sbody, 0)

        def pair(g, carry):
            sA = lax.rem(2 * g, 4)

            @pl.when(g < NPAIR - 1)
            def _():
                load_idx(2 * g + 2, lax.rem(sA + 2, 4))
                load_idx(2 * g + 3, lax.rem(sA + 3, 4))
            wait_idx(2 * g, sA)
            gidx(sA, 0)
            gA = fire_gathers(0)
            wait_idx(2 * g + 1, sA + 1)
            gidx(sA + 1, 1)
            gB = fire_gathers(1)
            gA[1].wait()
            gA[2].wait()
            compute_ex(0)
            dA = pltpu.async_copy(ex_v.at[0], den_s.at[dsc_v.at[0]], dsem.at[0], add=True)
            gA[0].wait()
            scale(0)
            sA = pltpu.async_copy(rows_v.at[0], u_s.at[dsc_v.at[0]], ssem.at[0], add=True)
            gB[1].wait()
            gB[2].wait()
            compute_ex(1)
            dB = pltpu.async_copy(ex_v.at[1], den_s.at[dsc_v.at[1]], dsem.at[1], add=True)
            gB[0].wait()
            scale(1)
            sB = pltpu.async_copy(rows_v.at[1], u_s.at[dsc_v.at[1]], ssem.at[1], add=True)
            dA.wait()
            sA.wait()
            dB.wait()
            sB.wait()
            return carry

        load_idx(0, 0)
        load_idx(1, 1)
        lax.fori_loop(0, NPAIR, pair, 0)
        if NCH % 2 == 1:
            load_idx(NCH - 1, 0)
            wait_idx(NCH - 1, 0)
            gidx(0, 0)
            gA = fire_gathers(0)
            gA[1].wait()
            gA[2].wait()
            compute_ex(0)
            pltpu.sync_copy(ex_v.at[0], den_s.at[dsc_v.at[0]], add=True)
            gA[0].wait()
            scale(0)
            pltpu.sync_copy(rows_v.at[0], u_s.at[dsc_v.at[0]], add=True)
        plsc.subcore_barrier()

        @pl.when(cid == 0)
        def _():
            pltpu.sync_copy(u_s.at[pl.ds(sid * URT, URT)], u0.at[pl.ds(sid * URT, URT)])
            pltpu.sync_copy(den_s.at[pl.ds(sid * DCH, DCH)], d0.at[pl.ds(sid * DCH, DCH)])

        @pl.when(cid == 1)
        def _():
            pltpu.sync_copy(u_s.at[pl.ds(sid * URT, URT)], u1.at[pl.ds(sid * URT, URT)])
            pltpu.sync_copy(den_s.at[pl.ds(sid * DCH, DCH)], d1.at[pl.ds(sid * DCH, DCH)])

    return pl.kernel(
        body,
        out_type=(
            jax.ShapeDtypeStruct((NU, cp), jnp.float32),
            jax.ShapeDtypeStruct((NU, cp), jnp.float32),
            jax.ShapeDtypeStruct((NU,), jnp.float32),
            jax.ShapeDtypeStruct((NU,), jnp.float32),
        ),
        mesh=mesh,
        scratch_types=[
            pltpu.VMEM((4, CH), jnp.int32),
            pltpu.VMEM((4, CH), jnp.int32),
            pltpu.VMEM((4, CH), jnp.int32),
            pltpu.VMEM((2, CH), jnp.int32),
            pltpu.VMEM((2, CH), jnp.int32),
            pltpu.VMEM((2, CH), jnp.int32),
            pltpu.VMEM((2, CH), jnp.float32),
            pltpu.VMEM((2, CH), jnp.float32),
            pltpu.VMEM((2, CH), jnp.float32),
            pltpu.VMEM((2, CH, cp), jnp.float32),
            pltpu.VMEM((16,), jnp.float32),
            pltpu.VMEM((zr, cp), jnp.float32),
            pltpu.VMEM((DCH,), jnp.float32),
            pltpu.VMEM_SHARED((NU, cp), jnp.float32),
            pltpu.VMEM_SHARED((NU,), jnp.float32),
            pltpu.VMEM_SHARED((tbl,), jnp.float32),
            pltpu.VMEM_SHARED((tbl,), jnp.float32),
            pltpu.SemaphoreType.DMA((4,)),
            pltpu.SemaphoreType.DMA((2,)),
            pltpu.SemaphoreType.DMA((2,)),
            pltpu.SemaphoreType.DMA((2,)),
            pltpu.SemaphoreType.DMA((2,)),
            pltpu.SemaphoreType.DMA((2,)),
        ],
    )


@functools.lru_cache(maxsize=None)
def _make_edge3(tbl, tstride):
    cp = 128
    mesh = plsc.VectorSubcoreMesh(core_axis_name="c", subcore_axis_name="s")
    stg = tbl // NS
    zr = 64
    nvec = cp // 16

    def body(qt0_h, qt1_h, kt0_h, kt1_h, src_h, dst_h, typ_h, xr0_h, xr1_h,
             m0_h, m1_h,
             u0, u1, d0, d1,
             src_v, dst_v, typ_v, gsrc_v, gdst_v, dsc_v, q_v, k_v, ex_v, rows_v,
             m_v, zb_v, zd_v, u_s, den_s, qtab_s, ktab_s,
             isem, rsem, qsem, ksem, dsem, ssem):
        cid = lax.axis_index("c")
        sid = lax.axis_index("s")

        @pl.when(cid == 0)
        def _():
            pltpu.sync_copy(qt0_h.at[pl.ds(sid * stg, stg)], qtab_s.at[pl.ds(sid * stg, stg)])
            pltpu.sync_copy(kt0_h.at[pl.ds(sid * stg, stg)], ktab_s.at[pl.ds(sid * stg, stg)])
            pltpu.sync_copy(m0_h, m_v)

        @pl.when(cid == 1)
        def _():
            pltpu.sync_copy(qt1_h.at[pl.ds(sid * stg, stg)], qtab_s.at[pl.ds(sid * stg, stg)])
            pltpu.sync_copy(kt1_h.at[pl.ds(sid * stg, stg)], ktab_s.at[pl.ds(sid * stg, stg)])
            pltpu.sync_copy(m1_h, m_v)

        def zrow(i, c):
            for j in range(nvec):
                zb_v[i, pl.ds(j * 16, 16)] = jnp.zeros((16,), jnp.float32)
            return c
        lax.fori_loop(0, zr, zrow, 0)

        def zden(i, c):
            zd_v[pl.ds(i * 16, 16)] = jnp.zeros((16,), jnp.float32)
            return c
        lax.fori_loop(0, DCH // 16, zden, 0)

        for j in range(URT // zr):
            pltpu.sync_copy(zb_v, u_s.at[pl.ds(sid * URT + j * zr, zr)])
        pltpu.sync_copy(zd_v, den_s.at[pl.ds(sid * DCH, DCH)])

        plsc.subcore_barrier()
        mv = m_v[...]

        def load_idx(c, sl4):
            base = sid * ET2 + c * CH
            pltpu.async_copy(src_h.at[pl.ds(base, CH)], src_v.at[sl4], isem.at[sl4])
            pltpu.async_copy(dst_h.at[pl.ds(base, CH)], dst_v.at[sl4], isem.at[sl4])
            pltpu.async_copy(typ_h.at[pl.ds(base, CH)], typ_v.at[sl4], isem.at[sl4])

        def wait_idx(c, sl4):
            base = sid * ET2 + c * CH
            pltpu.make_async_copy(src_h.at[pl.ds(base, CH)], src_v.at[sl4], isem.at[sl4]).wait()
            pltpu.make_async_copy(dst_h.at[pl.ds(base, CH)], dst_v.at[sl4], isem.at[sl4]).wait()
            pltpu.make_async_copy(typ_h.at[pl.ds(base, CH)], typ_v.at[sl4], isem.at[sl4]).wait()

        def gidx(sl4, s):
            for i in range(CH // 16):
                sl = pl.ds(i * 16, 16)
                tv = typ_v[sl4, sl]
                gsrc_v[s, sl] = tv * tstride + src_v[sl4, sl]
                gdst_v[s, sl] = tv * tstride + dst_v[sl4, sl]
                dsc_v[s, sl] = dst_v[sl4, sl]

        def compute_ex(s):
            for i in range(CH // 16):
                sl = pl.ds(i * 16, 16)
                z = q_v[s, sl] + k_v[s, sl]
                z = jnp.maximum(z, 0.2 * z)
                ex_v[s, sl] = jnp.exp(z - mv)

        def scale(s):
            def sbody(g, c2):
                exg = ex_v[s, pl.ds(g * 16, 16)]
                for i in range(16):
                    sc = exg[i]
                    for j in range(nvec):
                        slj = pl.ds(j * 16, 16)
                        rows_v[s, g * 16 + i, slj] = rows_v[s, g * 16 + i, slj] * sc
                return c2
            lax.fori_loop(0, CH // 16, sbody, 0)

        def sweep(xr_h):
            def fire_gathers(s):
                return (pltpu.async_copy(xr_h.at[gsrc_v.at[s]], rows_v.at[s], rsem.at[s]),
                        pltpu.async_copy(qtab_s.at[gdst_v.at[s]], q_v.at[s], qsem.at[s]),
                        pltpu.async_copy(ktab_s.at[gsrc_v.at[s]], k_v.at[s], ksem.at[s]))

            def pair(g, carry):
                sA = lax.rem(2 * g, 4)

                @pl.when(g < NPAIR2 - 1)
                def _():
                    load_idx(2 * g + 2, lax.rem(sA + 2, 4))
                    load_idx(2 * g + 3, lax.rem(sA + 3, 4))
                wait_idx(2 * g, sA)
                gidx(sA, 0)
                gA = fire_gathers(0)
                wait_idx(2 * g + 1, sA + 1)
                gidx(sA + 1, 1)
                gB = fire_gathers(1)
                gA[1].wait()
                gA[2].wait()
                compute_ex(0)
                dA = pltpu.async_copy(ex_v.at[0], den_s.at[dsc_v.at[0]], dsem.at[0], add=True)
                gA[0].wait()
                scale(0)
                sA_ = pltpu.async_copy(rows_v.at[0], u_s.at[dsc_v.at[0]], ssem.at[0], add=True)
                gB[1].wait()
                gB[2].wait()
                compute_ex(1)
                dB = pltpu.async_copy(ex_v.at[1], den_s.at[dsc_v.at[1]], dsem.at[1], add=True)
                gB[0].wait()
                scale(1)
                sB = pltpu.async_copy(rows_v.at[1], u_s.at[dsc_v.at[1]], ssem.at[1], add=True)
                dA.wait()
                sA_.wait()
                dB.wait()
                sB.wait()
                return carry

            load_idx(0, 0)
            load_idx(1, 1)
            lax.fori_loop(0, NPAIR2, pair, 0)

        @pl.when(cid == 0)
        def _():
            sweep(xr0_h)

        @pl.when(cid == 1)
        def _():
            sweep(xr1_h)

        plsc.subcore_barrier()

        @pl.when(cid == 0)
        def _():
            pltpu.sync_copy(u_s.at[pl.ds(sid * URT, URT)], u0.at[pl.ds(sid * URT, URT)])
            pltpu.sync_copy(den_s.at[pl.ds(sid * DCH, DCH)], d0.at[pl.ds(sid * DCH, DCH)])

        @pl.when(cid == 1)
        def _():
            pltpu.sync_copy(u_s.at[pl.ds(sid * URT, URT)], u1.at[pl.ds(sid * URT, URT)])
            pltpu.sync_copy(den_s.at[pl.ds(sid * DCH, DCH)], d1.at[pl.ds(sid * DCH, DCH)])

    return pl.kernel(
        body,
        out_type=(
            jax.ShapeDtypeStruct((NU, cp), jnp.float32),
            jax.ShapeDtypeStruct((NU, cp), jnp.float32),
            jax.ShapeDtypeStruct((NU,), jnp.float32),
            jax.ShapeDtypeStruct((NU,), jnp.float32),
        ),
        mesh=mesh,
        scratch_types=[
            pltpu.VMEM((4, CH), jnp.int32),
            pltpu.VMEM((4, CH), jnp.int32),
            pltpu.VMEM((4, CH), jnp.int32),
            pltpu.VMEM((2, CH), jnp.int32),
            pltpu.VMEM((2, CH), jnp.int32),
            pltpu.VMEM((2, CH), jnp.int32),
            pltpu.VMEM((2, CH), jnp.float32),
            pltpu.VMEM((2, CH), jnp.float32),
            pltpu.VMEM((2, CH), jnp.float32),
            pltpu.VMEM((2, CH, cp), jnp.float32),
            pltpu.VMEM((16,), jnp.float32),
            pltpu.VMEM((zr, cp), jnp.float32),
            pltpu.VMEM((DCH,), jnp.float32),
            pltpu.VMEM_SHARED((NU, cp), jnp.float32),
            pltpu.VMEM_SHARED((NU,), jnp.float32),
            pltpu.VMEM_SHARED((tbl,), jnp.float32),
            pltpu.VMEM_SHARED((tbl,), jnp.float32),
            pltpu.SemaphoreType.DMA((4,)),
            pltpu.SemaphoreType.DMA((2,)),
            pltpu.SemaphoreType.DMA((2,)),
            pltpu.SemaphoreType.DMA((2,)),
            pltpu.SemaphoreType.DMA((2,)),
            pltpu.SemaphoreType.DMA((2,)),
        ],
    )


@functools.lru_cache(maxsize=None)
def _make_edge2(tbl, tstride):
    cp = 128
    mesh = plsc.VectorSubcoreMesh(core_axis_name="c", subcore_axis_name="s")
    stg = tbl // NS
    zr = 64
    nvec = cp // 16

    def body(qt0_h, qt1_h, kt0_h, kt1_h, src_h, dst_h, typ_h, xr_h, m0_h, m1_h,
             u0, u1, d00, d01, d10, d11,
             src_v, dst_v, typ_v, gsrc_v, gdst_v, dsc_v, q0_v, q1_v, k0_v, k1_v,
             ex0_v, ex1_v, rows_v, m0_v, m1_v, zb_v, zd_v,
             u_s, den0_s, den1_s, qt0_s, qt1_s, kt0_s, kt1_s,
             isem, rsem, q0sem, q1sem, k0sem, k1sem, d0sem, d1sem, ssem):
        cid = lax.axis_index("c")
        sid = lax.axis_index("s")
        wid = cid * NS + sid

        sls = pl.ds(sid * stg, stg)
        pltpu.sync_copy(qt0_h.at[sls], qt0_s.at[sls])
        pltpu.sync_copy(qt1_h.at[sls], qt1_s.at[sls])
        pltpu.sync_copy(kt0_h.at[sls], kt0_s.at[sls])
        pltpu.sync_copy(kt1_h.at[sls], kt1_s.at[sls])
        pltpu.sync_copy(m0_h, m0_v)
        pltpu.sync_copy(m1_h, m1_v)

        def zrow(i, c):
            for j in range(nvec):
                zb_v[i, pl.ds(j * 16, 16)] = jnp.zeros((16,), jnp.float32)
            return c
        lax.fori_loop(0, zr, zrow, 0)

        def zden(i, c):
            zd_v[pl.ds(i * 16, 16)] = jnp.zeros((16,), jnp.float32)
            return c
        lax.fori_loop(0, DCH // 16, zden, 0)

        for j in range(URT // zr):
            pltpu.sync_copy(zb_v, u_s.at[pl.ds(sid * URT + j * zr, zr)])
        pltpu.sync_copy(zd_v, den0_s.at[pl.ds(sid * DCH, DCH)])
        pltpu.sync_copy(zd_v, den1_s.at[pl.ds(sid * DCH, DCH)])

        plsc.subcore_barrier()
        mv0 = m0_v[...]
        mv1 = m1_v[...]

        def load_idx(c, sl4):
            base = wid * ET + c * CH
            pltpu.async_copy(src_h.at[pl.ds(base, CH)], src_v.at[sl4], isem.at[sl4])
            pltpu.async_copy(dst_h.at[pl.ds(base, CH)], dst_v.at[sl4], isem.at[sl4])
            pltpu.async_copy(typ_h.at[pl.ds(base, CH)], typ_v.at[sl4], isem.at[sl4])

        def wait_idx(c, sl4):
            base = wid * ET + c * CH
            pltpu.make_async_copy(src_h.at[pl.ds(base, CH)], src_v.at[sl4], isem.at[sl4]).wait()
            pltpu.make_async_copy(dst_h.at[pl.ds(base, CH)], dst_v.at[sl4], isem.at[sl4]).wait()
            pltpu.make_async_copy(typ_h.at[pl.ds(base, CH)], typ_v.at[sl4], isem.at[sl4]).wait()

        def gidx(sl4, s):
            for i in range(CH // 16):
                sl = pl.ds(i * 16, 16)
                tv = typ_v[sl4, sl]
                gsrc_v[s, sl] = tv * tstride + src_v[sl4, sl]
                gdst_v[s, sl] = tv * tstride + dst_v[sl4, sl]
                dsc_v[s, sl] = dst_v[sl4, sl]

        def fire_gathers(s):
            return (pltpu.async_copy(xr_h.at[gsrc_v.at[s]], rows_v.at[s], rsem.at[s]),
                    pltpu.async_copy(qt0_s.at[gdst_v.at[s]], q0_v.at[s], q0sem.at[s]),
                    pltpu.async_copy(qt1_s.at[gdst_v.at[s]], q1_v.at[s], q1sem.at[s]),
                    pltpu.async_copy(kt0_s.at[gsrc_v.at[s]], k0_v.at[s], k0sem.at[s]),
                    pltpu.async_copy(kt1_s.at[gsrc_v.at[s]], k1_v.at[s], k1sem.at[s]))

        def compute_ex(s):
            for i in range(CH // 16):
                sl = pl.ds(i * 16, 16)
                z0 = q0_v[s, sl] + k0_v[s, sl]
                z0 = jnp.maximum(z0, 0.2 * z0)
                ex0_v[s, sl] = jnp.exp(z0 - mv0)
                z1 = q1_v[s, sl] + k1_v[s, sl]
                z1 = jnp.maximum(z1, 0.2 * z1)
                ex1_v[s, sl] = jnp.exp(z1 - mv1)

        def scale(s):
            def sbody(g, c2):
                exg0 = ex0_v[s, pl.ds(g * 16, 16)]
                exg1 = ex1_v[s, pl.ds(g * 16, 16)]
                for i in range(16):
                    s0 = exg0[i]
                    s1 = exg1[i]
                    for j in range(3):
                        slj = pl.ds(j * 16, 16)
                        rows_v[s, g * 16 + i, slj] = rows_v[s, g * 16 + i, slj] * s0
                    for j in range(3, 6):
                        slj = pl.ds(j * 16, 16)
                        rows_v[s, g * 16 + i, slj] = rows_v[s, g * 16 + i, slj] * s1
                return c2
            lax.fori_loop(0, CH // 16, sbody, 0)

        def pair(g, carry):
            sA = lax.rem(2 * g, 4)

            @pl.when(g < NPAIR - 1)
            def _():
                load_idx(2 * g + 2, lax.rem(sA + 2, 4))
                load_idx(2 * g + 3, lax.rem(sA + 3, 4))
            wait_idx(2 * g, sA)
            gidx(sA, 0)
            gA = fire_gathers(0)
            wait_idx(2 * g + 1, sA + 1)
            gidx(sA + 1, 1)
            gB = fire_gathers(1)
            for w in gA[1:]:
                w.wait()
            compute_ex(0)
            dA0 = pltpu.async_copy(ex0_v.at[0], den0_s.at[dsc_v.at[0]], d0sem.at[0], add=True)
            dA1 = pltpu.async_copy(ex1_v.at[0], den1_s.at[dsc_v.at[0]], d1sem.at[0], add=True)
            gA[0].wait()
            scale(0)
            sA = pltpu.async_copy(rows_v.at[0], u_s.at[dsc_v.at[0]], ssem.at[0], add=True)
            for w in gB[1:]:
                w.wait()
            compute_ex(1)
            dB0 = pltpu.async_copy(ex0_v.at[1], den0_s.at[dsc_v.at[1]], d0sem.at[1], add=True)
            dB1 = pltpu.async_copy(ex1_v.at[1], den1_s.at[dsc_v.at[1]], d1sem.at[1], add=True)
            gB[0].wait()
            scale(1)
            sB = pltpu.async_copy(rows_v.at[1], u_s.at[dsc_v.at[1]], ssem.at[1], add=True)
            dA0.wait()
            dA1.wait()
            sA.wait()
            dB0.wait()
            dB1.wait()
            sB.wait()
            return carry

        load_idx(0, 0)
        load_idx(1, 1)
        lax.fori_loop(0, NPAIR, pair, 0)
        if NCH % 2 == 1:
            load_idx(NCH - 1, 0)
            wait_idx(NCH - 1, 0)
            gidx(0, 0)
            gA = fire_gathers(0)
            for w in gA[1:]:
                w.wait()
            compute_ex(0)
            pltpu.sync_copy(ex0_v.at[0], den0_s.at[dsc_v.at[0]], add=True)
            pltpu.sync_copy(ex1_v.at[0], den1_s.at[dsc_v.at[0]], add=True)
            gA[0].wait()
            scale(0)
            pltpu.sync_copy(rows_v.at[0], u_s.at[dsc_v.at[0]], add=True)
        plsc.subcore_barrier()

        @pl.when(cid == 0)
        def _():
            pltpu.sync_copy(u_s.at[pl.ds(sid * URT, URT)], u0.at[pl.ds(sid * URT, URT)])
            pltpu.sync_copy(den0_s.at[pl.ds(sid * DCH, DCH)], d00.at[pl.ds(sid * DCH, DCH)])
            pltpu.sync_copy(den1_s.at[pl.ds(sid * DCH, DCH)], d10.at[pl.ds(sid * DCH, DCH)])

        @pl.when(cid == 1)
        def _():
            pltpu.sync_copy(u_s.at[pl.ds(sid * URT, URT)], u1.at[pl.ds(sid * URT, URT)])
            pltpu.sync_copy(den0_s.at[pl.ds(sid * DCH, DCH)], d01.at[pl.ds(sid * DCH, DCH)])
            pltpu.sync_copy(den1_s.at[pl.ds(sid * DCH, DCH)], d11.at[pl.ds(sid * DCH, DCH)])

    return pl.kernel(
        body,
        out_type=(
            jax.ShapeDtypeStruct((NU, cp), jnp.float32),
            jax.ShapeDtypeStruct((NU, cp), jnp.float32),
            jax.ShapeDtypeStruct((NU,), jnp.float32),
            jax.ShapeDtypeStruct((NU,), jnp.float32),
            jax.ShapeDtypeStruct((NU,), jnp.float32),
            jax.ShapeDtypeStruct((NU,), jnp.float32),
        ),
        mesh=mesh,
        scratch_types=[
            pltpu.VMEM((4, CH), jnp.int32),
            pltpu.VMEM((4, CH), jnp.int32),
            pltpu.VMEM((4, CH), jnp.int32),
            pltpu.VMEM((2, CH), jnp.int32),
            pltpu.VMEM((2, CH), jnp.int32),
            pltpu.VMEM((2, CH), jnp.int32),
            pltpu.VMEM((2, CH), jnp.float32),
            pltpu.VMEM((2, CH), jnp.float32),
            pltpu.VMEM((2, CH), jnp.float32),
            pltpu.VMEM((2, CH), jnp.float32),
            pltpu.VMEM((2, CH), jnp.float32),
            pltpu.VMEM((2, CH), jnp.float32),
            pltpu.VMEM((2, CH, cp), jnp.float32),
            pltpu.VMEM((16,), jnp.float32),
            pltpu.VMEM((16,), jnp.float32),
            pltpu.VMEM((zr, cp), jnp.float32),
            pltpu.VMEM((DCH,), jnp.float32),
            pltpu.VMEM_SHARED((NU, cp), jnp.float32),
            pltpu.VMEM_SHARED((NU,), jnp.float32),
            pltpu.VMEM_SHARED((NU,), jnp.float32),
            pltpu.VMEM_SHARED((tbl,), jnp.float32),
            pltpu.VMEM_SHARED((tbl,), jnp.float32),
            pltpu.VMEM_SHARED((tbl,), jnp.float32),
            pltpu.VMEM_SHARED((tbl,), jnp.float32),
            pltpu.SemaphoreType.DMA((4,)),
            pltpu.SemaphoreType.DMA((2,)),
            pltpu.SemaphoreType.DMA((2,)),
            pltpu.SemaphoreType.DMA((2,)),
            pltpu.SemaphoreType.DMA((2,)),
            pltpu.SemaphoreType.DMA((2,)),
            pltpu.SemaphoreType.DMA((2,)),
            pltpu.SemaphoreType.DMA((2,)),
            pltpu.SemaphoreType.DMA((2,)),
        ],
    )


RB = 400
NB = N // RB


@functools.lru_cache(maxsize=None)
def _make_post_rgat(cp):
    def body(u00, u01, u10, u11, d00, d01, d10, d11, b_ref, o_ref):
        den0 = d00[0, 0] + d01[0, 0] + 1e-16
        den1 = d10[0, 0] + d11[0, 0] + 1e-16
        num0 = u00[...] + u01[...]
        num1 = u10[...] + u11[...]
        o = 0.5 * (num0 / den0[:, None] + num1 / den1[:, None]) + b_ref[0][None, :]
        o_ref[...] = jnp.where(o > 0, o, jnp.exp(o) - 1.0)

    ub = pl.BlockSpec((RB, cp), lambda i: (i, 0))
    db = pl.BlockSpec((1, 1, RB), lambda i: (i, 0, 0))
    return pl.pallas_call(
        body,
        grid=(NB,),
        in_specs=[ub, ub, ub, ub, db, db, db, db,
                  pl.BlockSpec((1, cp), lambda i: (0, 0))],
        out_specs=ub,
        out_shape=jax.ShapeDtypeStruct((N, cp), jnp.float32),
    )


@functools.lru_cache(maxsize=None)
def _make_post_split(cp):
    def body(u0, u1, d0, d1, b_ref, o_ref):
        den0 = d0[0, 0] + 1e-16
        den1 = d1[0, 0] + 1e-16
        o = 0.5 * (u0[...] / den0[:, None] + u1[...] / den1[:, None]) + b_ref[0][None, :]
        o_ref[...] = jnp.where(o > 0, o, jnp.exp(o) - 1.0)

    ub = pl.BlockSpec((RB, cp), lambda i: (i, 0))
    db = pl.BlockSpec((1, 1, RB), lambda i: (i, 0, 0))
    return pl.pallas_call(
        body,
        grid=(NB,),
        in_specs=[ub, ub, db, db, pl.BlockSpec((1, cp), lambda i: (0, 0))],
        out_specs=ub,
        out_shape=jax.ShapeDtypeStruct((N, cp), jnp.float32),
    )


@functools.lru_cache(maxsize=None)
def _make_post_pair():
    def body(u0, u1, d00, d01, d10, d11, b_ref, o_ref):
        den0 = d00[0, 0] + d01[0, 0] + 1e-16
        den1 = d10[0, 0] + d11[0, 0] + 1e-16
        num = u0[...] + u1[...]
        o = 0.5 * (num[:, :48] / den0[:, None] + num[:, 48:96] / den1[:, None]) \
            + b_ref[0][None, :]
        o_ref[...] = jnp.where(o > 0, o, jnp.exp(o) - 1.0)

    ub = pl.BlockSpec((RB, 128), lambda i: (i, 0))
    db = pl.BlockSpec((1, 1, RB), lambda i: (i, 0, 0))
    return pl.pallas_call(
        body,
        grid=(NB,),
        in_specs=[ub, ub, db, db, db, db,
                  pl.BlockSpec((1, 48), lambda i: (0, 0))],
        out_specs=pl.BlockSpec((RB, 48), lambda i: (i, 0)),
        out_shape=jax.ShapeDtypeStruct((N, 48), jnp.float32),
    )


@functools.lru_cache(maxsize=None)
def _make_post_self(cp):
    def body(u0, u1, d0, d1, b_ref, o_ref):
        den = d0[0, 0] + d1[0, 0] + 1e-16
        num = u0[...] + u1[...]
        o = num / den[:, None] + b_ref[0][None, :]
        o_ref[...] = jnp.where(o > 0, o, jnp.exp(o) - 1.0)

    ub = pl.BlockSpec((RB, cp), lambda i: (i, 0))
    db = pl.BlockSpec((1, 1, RB), lambda i: (i, 0, 0))
    return pl.pallas_call(
        body,
        grid=(NB,),
        in_specs=[ub, ub, db, db, pl.BlockSpec((1, cp), lambda i: (0, 0))],
        out_specs=ub,
        out_shape=jax.ShapeDtypeStruct((N, cp), jnp.float32),
    )


def _leaky(z):
    return jnp.maximum(z, 0.2 * z)


def _pad_rows(a):
    return jnp.pad(a, ((0, NP - a.shape[0]), (0, 0)))


def _prep_w(w, q, k, heads, outc, cp):
    d = w.shape[1]
    wp = w.reshape(R, d, heads, outc).transpose(2, 0, 1, 3)
    qp = q.reshape(R, heads, outc).transpose(1, 0, 2)[:, :, None, :]
    kp = k.reshape(R, heads, outc).transpose(1, 0, 2)[:, :, None, :]
    if cp != outc:
        pad = ((0, 0), (0, 0), (0, 0), (0, cp - outc))
        wp = jnp.pad(wp, pad)
        qp = jnp.pad(qp, pad)
        kp = jnp.pad(kp, pad)
    return wp, qp, kp


def _rgat(xp, w, q, k, b, src, dst, typ, outc, cp):
    wp, qp, kp = _prep_w(w, q, k, HEADS, outc, cp)
    xr, qt, kt, qm, km = _make_pre(HEADS, R, cp)(xp, wp, qp, kp)
    xrh = xr.reshape(HEADS, R * NP, cp)
    qtab = qt.reshape(HEADS, R * NP)
    ktab = kt.reshape(HEADS, R * NP)
    mh = _leaky(qm.reshape(HEADS, -1).max(axis=1) + km.reshape(HEADS, -1).max(axis=1))
    m160 = jnp.broadcast_to(jnp.reshape(mh[0], (1,)), (16,))
    m161 = jnp.broadcast_to(jnp.reshape(mh[1], (1,)), (16,))
    u0, u1, d0, d1 = _make_edge3(R * NP, NP)(
        qtab[0], qtab[1], ktab[0], ktab[1], src, dst, typ, xrh[0], xrh[1],
        m160, m161)
    bp = jnp.pad(b, (0, cp - b.shape[0]))[None, :]
    return _make_post_split(cp)(u0, u1, d0[:N].reshape(NB, 1, RB),
                                d1[:N].reshape(NB, 1, RB), bp)


def _rgat_pair(xp, w, q, k, b, src, dst, typ):
    d = w.shape[1]
    wp4 = jnp.zeros((R, d, 128), jnp.float32)
    wp4 = wp4.at[:, :, 0:OUTS].set(w.reshape(R, d, HEADS, OUTS)[:, :, 0])
    wp4 = wp4.at[:, :, 48:48 + OUTS].set(w.reshape(R, d, HEADS, OUTS)[:, :, 1])
    qp4 = jnp.zeros((HEADS, R, 128), jnp.float32)
    qp4 = qp4.at[0, :, 0:OUTS].set(q.reshape(R, HEADS, OUTS)[:, 0])
    qp4 = qp4.at[1, :, 48:48 + OUTS].set(q.reshape(R, HEADS, OUTS)[:, 1])
    kp4 = jnp.zeros((HEADS, R, 128), jnp.float32)
    kp4 = kp4.at[0, :, 0:OUTS].set(k.reshape(R, HEADS, OUTS)[:, 0])
    kp4 = kp4.at[1, :, 48:48 + OUTS].set(k.reshape(R, HEADS, OUTS)[:, 1])
    wp = jnp.broadcast_to(wp4[None], (HEADS, R, d, 128))
    xr, qt, kt, qm, km = _make_pre(HEADS, R, 128)(xp, wp, qp4[:, :, None, :], kp4[:, :, None, :])
    xrh = xr.reshape(HEADS, R * NP, 128)
    qtab = qt.reshape(HEADS, R * NP)
    ktab = kt.reshape(HEADS, R * NP)
    mh = _leaky(qm.reshape(HEADS, -1).max(axis=1) + km.reshape(HEADS, -1).max(axis=1))
    m0 = jnp.broadcast_to(jnp.reshape(mh[0], (1,)), (16,))
    m1 = jnp.broadcast_to(jnp.reshape(mh[1], (1,)), (16,))
    u0, u1, d00, d01, d10, d11 = _make_edge2(R * NP, NP)(
        qtab[0], qtab[1], ktab[0], ktab[1], src, dst, typ, xrh[0], m0, m1)
    bp = jnp.pad(b, (0, 48 - OUTS))[None, :]
    return _make_post_pair()(
        u0, u1,
        d00[:N].reshape(NB, 1, RB), d01[:N].reshape(NB, 1, RB),
        d10[:N].reshape(NB, 1, RB), d11[:N].reshape(NB, 1, RB), bp)


def kernel(x, edge_index, edge_type, W1, Q1, K1, b1, WS, QS, KS, bS,
           W2, Q2, K2, b2, WT, QT, KT, bT, Wr, a_src, a_dst, br):
    src = edge_index[0]
    dst = edge_index[1]
    typ = edge_type
    xp = _pad_rows(x)

    x_s = _rgat(xp, W1, Q1, K1, b1, src, dst, typ, HID, HID)
    x_in = _rgat_pair(_pad_rows(x_s), WS, QS, KS, bS, src, dst, typ)[:, :OUTS]

    x_t = _rgat(xp, W2, Q2, K2, b2, src, dst, typ, HID, HID)
    x_out = _rgat_pair(_pad_rows(x_t), WT, QT, KT, bT, src, dst, typ)[:, :OUTS]

    wr = Wr[None, None]
    qp = a_dst[None, None, None, :]
    kp = a_src[None, None, None, :]
    xr, qt, kt, qm, km = _make_pre(1, 1, HID)(xp, wr, qp, kp)
    m0 = _leaky(qm.max() + km.max())
    m16 = jnp.broadcast_to(jnp.reshape(m0, (1,)), (16,))
    u0, u1, d0, d1 = _make_edge(NP, HID, 0)(
        qt.reshape(NP), kt.reshape(NP), src, dst, typ, xr.reshape(NP, HID), m16)
    x_self = _make_post_self(HID)(u0, u1, d0[:N].reshape(NB, 1, RB), d1[:N].reshape(NB, 1, RB),
                                  br[None, :])
    return (x_in, x_out, x_self)


# speedup vs baseline: 3.7255x; 1.0001x over previous
import functools

import jax
import jax.numpy as jnp
from jax import lax
from jax.experimental import pallas as pl
from jax.experimental.pallas import tpu as pltpu
from jax.experimental.pallas import tpu_sc as plsc

N = 10000
E = 320000
IN = 128
HEADS = 2
R = 4
HID = 128
OUTS = 33
CPS = 128
NP = 10240
TN = 256
NT = NP // TN

NC = 2
NS = 16
NW = NC * NS
ET = E // NW
CH = 80
NCH = ET // CH
NPAIR = NCH // 2
ET2 = E // NS
NCH2 = ET2 // CH
NPAIR2 = NCH2 // 2
NU = 10240
URT = NU // NS
DCH = NU // NS


@functools.lru_cache(maxsize=None)
def _make_pre(h_, r_, cp):
    def body(x_ref, w_ref, q_ref, k_ref, xr_ref, qt_ref, kt_ref, qm_ref, km_ref):
        xr = jnp.dot(x_ref[...], w_ref[0, 0], preferred_element_type=jnp.float32)
        xr_ref[0, 0] = xr
        qv = jnp.sum(xr * q_ref[0, 0, 0][None, :], axis=1)
        kv = jnp.sum(xr * k_ref[0, 0, 0][None, :], axis=1)
        qt_ref[0, 0] = qv
        kt_ref[0, 0] = kv
        qm_ref[0, 0] = jnp.full((TN,), jnp.max(qv), jnp.float32)
        km_ref[0, 0] = jnp.full((TN,), jnp.max(kv), jnp.float32)

    g = h_ * r_ * NT
    return pl.pallas_call(
        body,
        grid=(h_, r_, NT),
        in_specs=[
            pl.BlockSpec((TN, IN), lambda h, r, i: (i, 0)),
            pl.BlockSpec((1, 1, IN, cp), lambda h, r, i: (h, r, 0, 0)),
            pl.BlockSpec((1, 1, 1, cp), lambda h, r, i: (h, r, 0, 0)),
            pl.BlockSpec((1, 1, 1, cp), lambda h, r, i: (h, r, 0, 0)),
        ],
        out_specs=[
            pl.BlockSpec((1, 1, TN, cp), lambda h, r, i: (h, r, i, 0)),
            pl.BlockSpec((1, 1, TN), lambda h, r, i: ((h * r_ + r) * NT + i, 0, 0)),
            pl.BlockSpec((1, 1, TN), lambda h, r, i: ((h * r_ + r) * NT + i, 0, 0)),
            pl.BlockSpec((1, 1, TN), lambda h, r, i: ((h * r_ + r) * NT + i, 0, 0)),
            pl.BlockSpec((1, 1, TN), lambda h, r, i: ((h * r_ + r) * NT + i, 0, 0)),
        ],
        out_shape=[
            jax.ShapeDtypeStruct((h_, r_, NP, cp), jnp.float32),
            jax.ShapeDtypeStruct((g, 1, TN), jnp.float32),
            jax.ShapeDtypeStruct((g, 1, TN), jnp.float32),
            jax.ShapeDtypeStruct((g, 1, TN), jnp.float32),
            jax.ShapeDtypeStruct((g, 1, TN), jnp.float32),
        ],
    )


@functools.lru_cache(maxsize=None)
def _make_edge(tbl, cp, tstride):
    mesh = plsc.VectorSubcoreMesh(core_axis_name="c", subcore_axis_name="s")
    stg = tbl // NS
    zr = 64
    nvec = cp // 16

    def body(qtab_h, ktab_h, src_h, dst_h, typ_h, xr_h, m_h,
             u0, u1, d0, d1,
             src_v, dst_v, typ_v, gsrc_v, gdst_v, dsc_v, q_v, k_v, ex_v, rows_v,
             m_v, zb_v, zd_v, u_s, den_s, qtab_s, ktab_s,
             isem, rsem, qsem, ksem, dsem, ssem):
        cid = lax.axis_index("c")
        sid = lax.axis_index("s")
        wid = cid * NS + sid

        pltpu.sync_copy(qtab_h.at[pl.ds(sid * stg, stg)], qtab_s.at[pl.ds(sid * stg, stg)])
        pltpu.sync_copy(ktab_h.at[pl.ds(sid * stg, stg)], ktab_s.at[pl.ds(sid * stg, stg)])
        pltpu.sync_copy(m_h, m_v)

        def zrow(i, c):
            for j in range(nvec):
                zb_v[i, pl.ds(j * 16, 16)] = jnp.zeros((16,), jnp.float32)
            return c
        lax.fori_loop(0, zr, zrow, 0)

        def zden(i, c):
            zd_v[pl.ds(i * 16, 16)] = jnp.zeros((16,), jnp.float32)
            return c
        lax.fori_loop(0, DCH // 16, zden, 0)

        for j in range(URT // zr):
            pltpu.sync_copy(zb_v, u_s.at[pl.ds(sid * URT + j * zr, zr)])
        pltpu.sync_copy(zd_v, den_s.at[pl.ds(sid * DCH, DCH)])

        plsc.subcore_barrier()
        mv = m_v[...]

        def load_idx(c, sl4):
            base = wid * ET + c * CH
            pltpu.async_copy(src_h.at[pl.ds(base, CH)], src_v.at[sl4], isem.at[sl4])
            pltpu.async_copy(dst_h.at[pl.ds(base, CH)], dst_v.at[sl4], isem.at[sl4])
            pltpu.async_copy(typ_h.at[pl.ds(base, CH)], typ_v.at[sl4], isem.at[sl4])

        def wait_idx(c, sl4):
            base = wid * ET + c * CH
            pltpu.make_async_copy(src_h.at[pl.ds(base, CH)], src_v.at[sl4], isem.at[sl4]).wait()
            pltpu.make_async_copy(dst_h.at[pl.ds(base, CH)], dst_v.at[sl4], isem.at[sl4]).wait()
            pltpu.make_async_copy(typ_h.at[pl.ds(base, CH)], typ_v.at[sl4], isem.at[sl4]).wait()

        def gidx(sl4, s):
            for i in range(CH // 16):
                sl = pl.ds(i * 16, 16)
                tv = typ_v[sl4, sl]
                gsrc_v[s, sl] = tv * tstride + src_v[sl4, sl]
                gdst_v[s, sl] = tv * tstride + dst_v[sl4, sl]
                dsc_v[s, sl] = dst_v[sl4, sl]

        def fire_gathers(s):
            return (pltpu.async_copy(xr_h.at[gsrc_v.at[s]], rows_v.at[s], rsem.at[s]),
                    pltpu.async_copy(qtab_s.at[gdst_v.at[s]], q_v.at[s], qsem.at[s]),
                    pltpu.async_copy(ktab_s.at[gsrc_v.at[s]], k_v.at[s], ksem.at[s]))

        def compute_ex(s):
            for i in range(CH // 16):
                sl = pl.ds(i * 16, 16)
                z = q_v[s, sl] + k_v[s, sl]
                z = jnp.maximum(z, 0.2 * z)
                ex_v[s, sl] = jnp.exp(z - mv)

        def scale(s):
            def sbody(g, c2):
                exg = ex_v[s, pl.ds(g * 16, 16)]
                for i in range(16):
                    sc = exg[i]
                    for j in range(nvec):
                        slj = pl.ds(j * 16, 16)
                        rows_v[s, g * 16 + i, slj] = rows_v[s, g * 16 + i, slj] * sc
                return c2
            lax.fori_loop(0, CH // 16, sbody, 0)

        def pair(g, carry):
            sA = lax.rem(2 * g, 4)

            @pl.when(g < NPAIR - 1)
            def _():
                load_idx(2 * g + 2, lax.rem(sA + 2, 4))
                load_idx(2 * g + 3, lax.rem(sA + 3, 4))
            wait_idx(2 * g, sA)
            gidx(sA, 0)
            gA = fire_gathers(0)
            wait_idx(2 * g + 1, sA + 1)
            gidx(sA + 1, 1)
            gB = fire_gathers(1)
            gA[1].wait()
            gA[2].wait()
            compute_ex(0)
            dA = pltpu.async_copy(ex_v.at[0], den_s.at[dsc_v.at[0]], dsem.at[0], add=True)
            gA[0].wait()
            scale(0)
            sA = pltpu.async_copy(rows_v.at[0], u_s.at[dsc_v.at[0]], ssem.at[0], add=True)
            gB[1].wait()
            gB[2].wait()
            compute_ex(1)
            dB = pltpu.async_copy(ex_v.at[1], den_s.at[dsc_v.at[1]], dsem.at[1], add=True)
            gB[0].wait()
            scale(1)
            sB = pltpu.async_copy(rows_v.at[1], u_s.at[dsc_v.at[1]], ssem.at[1], add=True)
            dA.wait()
            sA.wait()
            dB.wait()
            sB.wait()
            return carry

        load_idx(0, 0)
        load_idx(1, 1)
        lax.fori_loop(0, NPAIR, pair, 0)
        if NCH % 2 == 1:
            load_idx(NCH - 1, 0)
            wait_idx(NCH - 1, 0)
            gidx(0, 0)
            gA = fire_gathers(0)
            gA[1].wait()
            gA[2].wait()
            compute_ex(0)
            pltpu.sync_copy(ex_v.at[0], den_s.at[dsc_v.at[0]], add=True)
            gA[0].wait()
            scale(0)
            pltpu.sync_copy(rows_v.at[0], u_s.at[dsc_v.at[0]], add=True)
        plsc.subcore_barrier()

        @pl.when(cid == 0)
        def _():
            pltpu.sync_copy(u_s.at[pl.ds(sid * URT, URT)], u0.at[pl.ds(sid * URT, URT)])
            pltpu.sync_copy(den_s.at[pl.ds(sid * DCH, DCH)], d0.at[pl.ds(sid * DCH, DCH)])

        @pl.when(cid == 1)
        def _():
            pltpu.sync_copy(u_s.at[pl.ds(sid * URT, URT)], u1.at[pl.ds(sid * URT, URT)])
            pltpu.sync_copy(den_s.at[pl.ds(sid * DCH, DCH)], d1.at[pl.ds(sid * DCH, DCH)])

    return pl.kernel(
        body,
        out_type=(
            jax.ShapeDtypeStruct((NU, cp), jnp.float32),
            jax.ShapeDtypeStruct((NU, cp), jnp.float32),
            jax.ShapeDtypeStruct((NU,), jnp.float32),
            jax.ShapeDtypeStruct((NU,), jnp.float32),
        ),
        mesh=mesh,
        scratch_types=[
            pltpu.VMEM((4, CH), jnp.int32),
            pltpu.VMEM((4, CH), jnp.int32),
            pltpu.VMEM((4, CH), jnp.int32),
            pltpu.VMEM((2, CH), jnp.int32),
            pltpu.VMEM((2, CH), jnp.int32),
            pltpu.VMEM((2, CH), jnp.int32),
            pltpu.VMEM((2, CH), jnp.float32),
            pltpu.VMEM((2, CH), jnp.float32),
            pltpu.VMEM((2, CH), jnp.float32),
            pltpu.VMEM((2, CH, cp), jnp.float32),
            pltpu.VMEM((16,), jnp.float32),
            pltpu.VMEM((zr, cp), jnp.float32),
            pltpu.VMEM((DCH,), jnp.float32),
            pltpu.VMEM_SHARED((NU, cp), jnp.float32),
            pltpu.VMEM_SHARED((NU,), jnp.float32),
            pltpu.VMEM_SHARED((tbl,), jnp.float32),
            pltpu.VMEM_SHARED((tbl,), jnp.float32),
            pltpu.SemaphoreType.DMA((4,)),
            pltpu.SemaphoreType.DMA((2,)),
            pltpu.SemaphoreType.DMA((2,)),
            pltpu.SemaphoreType.DMA((2,)),
            pltpu.SemaphoreType.DMA((2,)),
            pltpu.SemaphoreType.DMA((2,)),
        ],
    )


@functools.lru_cache(maxsize=None)
def _make_edge3(tbl, tstride):
    cp = 128
    mesh = plsc.VectorSubcoreMesh(core_axis_name="c", subcore_axis_name="s")
    stg = tbl // NS
    zr = 64
    nvec = cp // 16

    def body(qt0_h, qt1_h, kt0_h, kt1_h, src_h, dst_h, typ_h, xr0_h, xr1_h,
             m0_h, m1_h,
             u0, u1, d0, d1,
             src_v, dst_v, typ_v, gsrc_v, gdst_v, dsc_v, q_v, k_v, ex_v, rows_v,
             m_v, zb_v, zd_v, u_s, den_s, qtab_s, ktab_s,
             isem, rsem, qsem, ksem, dsem, ssem):
        cid = lax.axis_index("c")
        sid = lax.axis_index("s")

        @pl.when(cid == 0)
        def _():
            pltpu.sync_copy(qt0_h.at[pl.ds(sid * stg, stg)], qtab_s.at[pl.ds(sid * stg, stg)])
            pltpu.sync_copy(kt0_h.at[pl.ds(sid * stg, stg)], ktab_s.at[pl.ds(sid * stg, stg)])
            pltpu.sync_copy(m0_h, m_v)

        @pl.when(cid == 1)
        def _():
            pltpu.sync_copy(qt1_h.at[pl.ds(sid * stg, stg)], qtab_s.at[pl.ds(sid * stg, stg)])
            pltpu.sync_copy(kt1_h.at[pl.ds(sid * stg, stg)], ktab_s.at[pl.ds(sid * stg, stg)])
            pltpu.sync_copy(m1_h, m_v)

        def zrow(i, c):
            for j in range(nvec):
                zb_v[i, pl.ds(j * 16, 16)] = jnp.zeros((16,), jnp.float32)
            return c
        lax.fori_loop(0, zr, zrow, 0)

        def zden(i, c):
            zd_v[pl.ds(i * 16, 16)] = jnp.zeros((16,), jnp.float32)
            return c
        lax.fori_loop(0, DCH // 16, zden, 0)

        for j in range(URT // zr):
            pltpu.sync_copy(zb_v, u_s.at[pl.ds(sid * URT + j * zr, zr)])
        pltpu.sync_copy(zd_v, den_s.at[pl.ds(sid * DCH, DCH)])

        plsc.subcore_barrier()
        mv = m_v[...]

        def load_idx(c, sl4):
            base = sid * ET2 + c * CH
            pltpu.async_copy(src_h.at[pl.ds(base, CH)], src_v.at[sl4], isem.at[sl4])
            pltpu.async_copy(dst_h.at[pl.ds(base, CH)], dst_v.at[sl4], isem.at[sl4])
            pltpu.async_copy(typ_h.at[pl.ds(base, CH)], typ_v.at[sl4], isem.at[sl4])

        def wait_idx(c, sl4):
            base = sid * ET2 + c * CH
            pltpu.make_async_copy(src_h.at[pl.ds(base, CH)], src_v.at[sl4], isem.at[sl4]).wait()
            pltpu.make_async_copy(dst_h.at[pl.ds(base, CH)], dst_v.at[sl4], isem.at[sl4]).wait()
            pltpu.make_async_copy(typ_h.at[pl.ds(base, CH)], typ_v.at[sl4], isem.at[sl4]).wait()

        def gidx(sl4, s):
            for i in range(CH // 16):
                sl = pl.ds(i * 16, 16)
                tv = typ_v[sl4, sl]
                gsrc_v[s, sl] = tv * tstride + src_v[sl4, sl]
                gdst_v[s, sl] = tv * tstride + dst_v[sl4, sl]
                dsc_v[s, sl] = dst_v[sl4, sl]

        def compute_ex(s):
            for i in range(CH // 16):
                sl = pl.ds(i * 16, 16)
                z = q_v[s, sl] + k_v[s, sl]
                z = jnp.maximum(z, 0.2 * z)
                ex_v[s, sl] = jnp.exp(z - mv)

        def scale(s):
            def sbody(g, c2):
                exg = ex_v[s, pl.ds(g * 16, 16)]
                for i in range(16):
                    sc = exg[i]
                    for j in range(nvec):
                        slj = pl.ds(j * 16, 16)
                        rows_v[s, g * 16 + i, slj] = rows_v[s, g * 16 + i, slj] * sc
                return c2
            lax.fori_loop(0, CH // 16, sbody, 0)

        def sweep(xr_h):
            def fire_gathers(s):
                return (pltpu.async_copy(xr_h.at[gsrc_v.at[s]], rows_v.at[s], rsem.at[s]),
                        pltpu.async_copy(qtab_s.at[gdst_v.at[s]], q_v.at[s], qsem.at[s]),
                        pltpu.async_copy(ktab_s.at[gsrc_v.at[s]], k_v.at[s], ksem.at[s]))

            def pair(g, carry):
                sA = lax.rem(2 * g, 4)

                @pl.when(g < NPAIR2 - 1)
                def _():
                    load_idx(2 * g + 2, lax.rem(sA + 2, 4))
                    load_idx(2 * g + 3, lax.rem(sA + 3, 4))
                wait_idx(2 * g, sA)
                gidx(sA, 0)
                gA = fire_gathers(0)
                wait_idx(2 * g + 1, sA + 1)
                gidx(sA + 1, 1)
                gB = fire_gathers(1)
                gA[1].wait()
                gA[2].wait()
                compute_ex(0)
                dA = pltpu.async_copy(ex_v.at[0], den_s.at[dsc_v.at[0]], dsem.at[0], add=True)
                gA[0].wait()
                scale(0)
                sA_ = pltpu.async_copy(rows_v.at[0], u_s.at[dsc_v.at[0]], ssem.at[0], add=True)
                gB[1].wait()
                gB[2].wait()
                compute_ex(1)
                dB = pltpu.async_copy(ex_v.at[1], den_s.at[dsc_v.at[1]], dsem.at[1], add=True)
                gB[0].wait()
                scale(1)
                sB = pltpu.async_copy(rows_v.at[1], u_s.at[dsc_v.at[1]], ssem.at[1], add=True)
                dA.wait()
                sA_.wait()
                dB.wait()
                sB.wait()
                return carry

            load_idx(0, 0)
            load_idx(1, 1)
            lax.fori_loop(0, NPAIR2, pair, 0)

        @pl.when(cid == 0)
        def _():
            sweep(xr0_h)

        @pl.when(cid == 1)
        def _():
            sweep(xr1_h)

        plsc.subcore_barrier()

        @pl.when(cid == 0)
        def _():
            pltpu.sync_copy(u_s.at[pl.ds(sid * URT, URT)], u0.at[pl.ds(sid * URT, URT)])
            pltpu.sync_copy(den_s.at[pl.ds(sid * DCH, DCH)], d0.at[pl.ds(sid * DCH, DCH)])

        @pl.when(cid == 1)
        def _():
            pltpu.sync_copy(u_s.at[pl.ds(sid * URT, URT)], u1.at[pl.ds(sid * URT, URT)])
            pltpu.sync_copy(den_s.at[pl.ds(sid * DCH, DCH)], d1.at[pl.ds(sid * DCH, DCH)])

    return pl.kernel(
        body,
        out_type=(
            jax.ShapeDtypeStruct((NU, cp), jnp.float32),
            jax.ShapeDtypeStruct((NU, cp), jnp.float32),
            jax.ShapeDtypeStruct((NU,), jnp.float32),
            jax.ShapeDtypeStruct((NU,), jnp.float32),
        ),
        mesh=mesh,
        scratch_types=[
            pltpu.VMEM((4, CH), jnp.int32),
            pltpu.VMEM((4, CH), jnp.int32),
            pltpu.VMEM((4, CH), jnp.int32),
            pltpu.VMEM((2, CH), jnp.int32),
            pltpu.VMEM((2, CH), jnp.int32),
            pltpu.VMEM((2, CH), jnp.int32),
            pltpu.VMEM((2, CH), jnp.float32),
            pltpu.VMEM((2, CH), jnp.float32),
            pltpu.VMEM((2, CH), jnp.float32),
            pltpu.VMEM((2, CH, cp), jnp.float32),
            pltpu.VMEM((16,), jnp.float32),
            pltpu.VMEM((zr, cp), jnp.float32),
            pltpu.VMEM((DCH,), jnp.float32),
            pltpu.VMEM_SHARED((NU, cp), jnp.float32),
            pltpu.VMEM_SHARED((NU,), jnp.float32),
            pltpu.VMEM_SHARED((tbl,), jnp.float32),
            pltpu.VMEM_SHARED((tbl,), jnp.float32),
            pltpu.SemaphoreType.DMA((4,)),
            pltpu.SemaphoreType.DMA((2,)),
            pltpu.SemaphoreType.DMA((2,)),
            pltpu.SemaphoreType.DMA((2,)),
            pltpu.SemaphoreType.DMA((2,)),
            pltpu.SemaphoreType.DMA((2,)),
        ],
    )


@functools.lru_cache(maxsize=None)
def _make_edge2(tbl, tstride):
    cp = 128
    mesh = plsc.VectorSubcoreMesh(core_axis_name="c", subcore_axis_name="s")
    stg = tbl // NS
    zr = 64
    nvec = cp // 16

    def body(qt0_h, qt1_h, kt0_h, kt1_h, src_h, dst_h, typ_h, xr_h, m0_h, m1_h,
             u0, u1, d00, d01, d10, d11,
             src_v, dst_v, typ_v, gsrc_v, gdst_v, dsc_v, q0_v, q1_v, k0_v, k1_v,
             ex0_v, ex1_v, rows_v, m0_v, m1_v, zb_v, zd_v,
             u_s, den0_s, den1_s, qt0_s, qt1_s, kt0_s, kt1_s,
             isem, rsem, q0sem, q1sem, k0sem, k1sem, d0sem, d1sem, ssem):
        cid = lax.axis_index("c")
        sid = lax.axis_index("s")
        wid = cid * NS + sid

        sls = pl.ds(sid * stg, stg)
        pltpu.sync_copy(qt0_h.at[sls], qt0_s.at[sls])
        pltpu.sync_copy(qt1_h.at[sls], qt1_s.at[sls])
        pltpu.sync_copy(kt0_h.at[sls], kt0_s.at[sls])
        pltpu.sync_copy(kt1_h.at[sls], kt1_s.at[sls])
        pltpu.sync_copy(m0_h, m0_v)
        pltpu.sync_copy(m1_h, m1_v)

        def zrow(i, c):
            for j in range(nvec):
                zb_v[i, pl.ds(j * 16, 16)] = jnp.zeros((16,), jnp.float32)
            return c
        lax.fori_loop(0, zr, zrow, 0)

        def zden(i, c):
            zd_v[pl.ds(i * 16, 16)] = jnp.zeros((16,), jnp.float32)
            return c
        lax.fori_loop(0, DCH // 16, zden, 0)

        for j in range(URT // zr):
            pltpu.sync_copy(zb_v, u_s.at[pl.ds(sid * URT + j * zr, zr)])
        pltpu.sync_copy(zd_v, den0_s.at[pl.ds(sid * DCH, DCH)])
        pltpu.sync_copy(zd_v, den1_s.at[pl.ds(sid * DCH, DCH)])

        plsc.subcore_barrier()
        mv0 = m0_v[...]
        mv1 = m1_v[...]

        def load_idx(c, sl4):
            base = wid * ET + c * CH
            pltpu.async_copy(src_h.at[pl.ds(base, CH)], src_v.at[sl4], isem.at[sl4])
            pltpu.async_copy(dst_h.at[pl.ds(base, CH)], dst_v.at[sl4], isem.at[sl4])
            pltpu.async_copy(typ_h.at[pl.ds(base, CH)], typ_v.at[sl4], isem.at[sl4])

        def wait_idx(c, sl4):
            base = wid * ET + c * CH
            pltpu.make_async_copy(src_h.at[pl.ds(base, CH)], src_v.at[sl4], isem.at[sl4]).wait()
            pltpu.make_async_copy(dst_h.at[pl.ds(base, CH)], dst_v.at[sl4], isem.at[sl4]).wait()
            pltpu.make_async_copy(typ_h.at[pl.ds(base, CH)], typ_v.at[sl4], isem.at[sl4]).wait()

        def gidx(sl4, s):
            for i in range(CH // 16):
                sl = pl.ds(i * 16, 16)
                tv = typ_v[sl4, sl]
                gsrc_v[s, sl] = tv * tstride + src_v[sl4, sl]
                gdst_v[s, sl] = tv * tstride + dst_v[sl4, sl]
                dsc_v[s, sl] = dst_v[sl4, sl]

        def fire_gathers(s):
            return (pltpu.async_copy(xr_h.at[gsrc_v.at[s]], rows_v.at[s], rsem.at[s]),
                    pltpu.async_copy(qt0_s.at[gdst_v.at[s]], q0_v.at[s], q0sem.at[s]),
                    pltpu.async_copy(qt1_s.at[gdst_v.at[s]], q1_v.at[s], q1sem.at[s]),
                    pltpu.async_copy(kt0_s.at[gsrc_v.at[s]], k0_v.at[s], k0sem.at[s]),
                    pltpu.async_copy(kt1_s.at[gsrc_v.at[s]], k1_v.at[s], k1sem.at[s]))

        def compute_ex(s):
            for i in range(CH // 16):
                sl = pl.ds(i * 16, 16)
                z0 = q0_v[s, sl] + k0_v[s, sl]
                z0 = jnp.maximum(z0, 0.2 * z0)
                ex0_v[s, sl] = jnp.exp(z0 - mv0)
                z1 = q1_v[s, sl] + k1_v[s, sl]
                z1 = jnp.maximum(z1, 0.2 * z1)
                ex1_v[s, sl] = jnp.exp(z1 - mv1)

        def scale(s):
            def sbody(g, c2):
                exg0 = ex0_v[s, pl.ds(g * 16, 16)]
                exg1 = ex1_v[s, pl.ds(g * 16, 16)]
                for i in range(16):
                    s0 = exg0[i]
                    s1 = exg1[i]
                    for j in range(3):
                        slj = pl.ds(j * 16, 16)
                        rows_v[s, g * 16 + i, slj] = rows_v[s, g * 16 + i, slj] * s0
                    for j in range(3, 6):
                        slj = pl.ds(j * 16, 16)
                        rows_v[s, g * 16 + i, slj] = rows_v[s, g * 16 + i, slj] * s1
                return c2
            lax.fori_loop(0, CH // 16, sbody, 0)

        def pair(g, carry):
            sA = lax.rem(2 * g, 4)

            @pl.when(g < NPAIR - 1)
            def _():
                load_idx(2 * g + 2, lax.rem(sA + 2, 4))
                load_idx(2 * g + 3, lax.rem(sA + 3, 4))
            wait_idx(2 * g, sA)
            gidx(sA, 0)
            gA = fire_gathers(0)
            wait_idx(2 * g + 1, sA + 1)
            gidx(sA + 1, 1)
            gB = fire_gathers(1)
            for w in gA[1:]:
                w.wait()
            compute_ex(0)
            dA0 = pltpu.async_copy(ex0_v.at[0], den0_s.at[dsc_v.at[0]], d0sem.at[0], add=True)
            dA1 = pltpu.async_copy(ex1_v.at[0], den1_s.at[dsc_v.at[0]], d1sem.at[0], add=True)
            gA[0].wait()
            scale(0)
            sA = pltpu.async_copy(rows_v.at[0], u_s.at[dsc_v.at[0]], ssem.at[0], add=True)
            for w in gB[1:]:
                w.wait()
            compute_ex(1)
            dB0 = pltpu.async_copy(ex0_v.at[1], den0_s.at[dsc_v.at[1]], d0sem.at[1], add=True)
            dB1 = pltpu.async_copy(ex1_v.at[1], den1_s.at[dsc_v.at[1]], d1sem.at[1], add=True)
            gB[0].wait()
            scale(1)
            sB = pltpu.async_copy(rows_v.at[1], u_s.at[dsc_v.at[1]], ssem.at[1], add=True)
            dA0.wait()
            dA1.wait()
            sA.wait()
            dB0.wait()
            dB1.wait()
            sB.wait()
            return carry

        load_idx(0, 0)
        load_idx(1, 1)
        lax.fori_loop(0, NPAIR, pair, 0)
        if NCH % 2 == 1:
            load_idx(NCH - 1, 0)
            wait_idx(NCH - 1, 0)
            gidx(0, 0)
            gA = fire_gathers(0)
            for w in gA[1:]:
                w.wait()
            compute_ex(0)
            pltpu.sync_copy(ex0_v.at[0], den0_s.at[dsc_v.at[0]], add=True)
            pltpu.sync_copy(ex1_v.at[0], den1_s.at[dsc_v.at[0]], add=True)
            gA[0].wait()
            scale(0)
            pltpu.sync_copy(rows_v.at[0], u_s.at[dsc_v.at[0]], add=True)
        plsc.subcore_barrier()

        @pl.when(cid == 0)
        def _():
            pltpu.sync_copy(u_s.at[pl.ds(sid * URT, URT)], u0.at[pl.ds(sid * URT, URT)])
            pltpu.sync_copy(den0_s.at[pl.ds(sid * DCH, DCH)], d00.at[pl.ds(sid * DCH, DCH)])
            pltpu.sync_copy(den1_s.at[pl.ds(sid * DCH, DCH)], d10.at[pl.ds(sid * DCH, DCH)])

        @pl.when(cid == 1)
        def _():
            pltpu.sync_copy(u_s.at[pl.ds(sid * URT, URT)], u1.at[pl.ds(sid * URT, URT)])
            pltpu.sync_copy(den0_s.at[pl.ds(sid * DCH, DCH)], d01.at[pl.ds(sid * DCH, DCH)])
            pltpu.sync_copy(den1_s.at[pl.ds(sid * DCH, DCH)], d11.at[pl.ds(sid * DCH, DCH)])

    return pl.kernel(
        body,
        out_type=(
            jax.ShapeDtypeStruct((NU, cp), jnp.float32),
            jax.ShapeDtypeStruct((NU, cp), jnp.float32),
            jax.ShapeDtypeStruct((NU,), jnp.float32),
            jax.ShapeDtypeStruct((NU,), jnp.float32),
            jax.ShapeDtypeStruct((NU,), jnp.float32),
            jax.ShapeDtypeStruct((NU,), jnp.float32),
        ),
        mesh=mesh,
        scratch_types=[
            pltpu.VMEM((4, CH), jnp.int32),
            pltpu.VMEM((4, CH), jnp.int32),
            pltpu.VMEM((4, CH), jnp.int32),
            pltpu.VMEM((2, CH), jnp.int32),
            pltpu.VMEM((2, CH), jnp.int32),
            pltpu.VMEM((2, CH), jnp.int32),
            pltpu.VMEM((2, CH), jnp.float32),
            pltpu.VMEM((2, CH), jnp.float32),
            pltpu.VMEM((2, CH), jnp.float32),
            pltpu.VMEM((2, CH), jnp.float32),
            pltpu.VMEM((2, CH), jnp.float32),
            pltpu.VMEM((2, CH), jnp.float32),
            pltpu.VMEM((2, CH, cp), jnp.float32),
            pltpu.VMEM((16,), jnp.float32),
            pltpu.VMEM((16,), jnp.float32),
            pltpu.VMEM((zr, cp), jnp.float32),
            pltpu.VMEM((DCH,), jnp.float32),
            pltpu.VMEM_SHARED((NU, cp), jnp.float32),
            pltpu.VMEM_SHARED((NU,), jnp.float32),
            pltpu.VMEM_SHARED((NU,), jnp.float32),
            pltpu.VMEM_SHARED((tbl,), jnp.float32),
            pltpu.VMEM_SHARED((tbl,), jnp.float32),
            pltpu.VMEM_SHARED((tbl,), jnp.float32),
            pltpu.VMEM_SHARED((tbl,), jnp.float32),
            pltpu.SemaphoreType.DMA((4,)),
            pltpu.SemaphoreType.DMA((2,)),
            pltpu.SemaphoreType.DMA((2,)),
            pltpu.SemaphoreType.DMA((2,)),
            pltpu.SemaphoreType.DMA((2,)),
            pltpu.SemaphoreType.DMA((2,)),
            pltpu.SemaphoreType.DMA((2,)),
            pltpu.SemaphoreType.DMA((2,)),
            pltpu.SemaphoreType.DMA((2,)),
        ],
    )


RB = 400
NB = N // RB


@functools.lru_cache(maxsize=None)
def _make_post_rgat(cp):
    def body(u00, u01, u10, u11, d00, d01, d10, d11, b_ref, o_ref):
        den0 = d00[0, 0] + d01[0, 0] + 1e-16
        den1 = d10[0, 0] + d11[0, 0] + 1e-16
        num0 = u00[...] + u01[...]
        num1 = u10[...] + u11[...]
        o = 0.5 * (num0 / den0[:, None] + num1 / den1[:, None]) + b_ref[0][None, :]
        o_ref[...] = jnp.where(o > 0, o, jnp.exp(o) - 1.0)

    ub = pl.BlockSpec((RB, cp), lambda i: (i, 0))
    db = pl.BlockSpec((1, 1, RB), lambda i: (i, 0, 0))
    return pl.pallas_call(
        body,
        grid=(NB,),
        in_specs=[ub, ub, ub, ub, db, db, db, db,
                  pl.BlockSpec((1, cp), lambda i: (0, 0))],
        out_specs=ub,
        out_shape=jax.ShapeDtypeStruct((N, cp), jnp.float32),
    )


@functools.lru_cache(maxsize=None)
def _make_post_split(cp):
    def body(u0, u1, d0, d1, b_ref, o_ref):
        den0 = d0[0, 0] + 1e-16
        den1 = d1[0, 0] + 1e-16
        o = 0.5 * (u0[...] / den0[:, None] + u1[...] / den1[:, None]) + b_ref[0][None, :]
        o_ref[...] = jnp.where(o > 0, o, jnp.exp(o) - 1.0)

    ub = pl.BlockSpec((RB, cp), lambda i: (i, 0))
    db = pl.BlockSpec((1, 1, RB), lambda i: (i, 0, 0))
    return pl.pallas_call(
        body,
        grid=(NB,),
        in_specs=[ub, ub, db, db, pl.BlockSpec((1, cp), lambda i: (0, 0))],
        out_specs=ub,
        out_shape=jax.ShapeDtypeStruct((N, cp), jnp.float32),
    )


@functools.lru_cache(maxsize=None)
def _make_post_pair():
    def body(u0, u1, d00, d01, d10, d11, b_ref, o_ref):
        den0 = d00[0, 0] + d01[0, 0] + 1e-16
        den1 = d10[0, 0] + d11[0, 0] + 1e-16
        num = u0[...] + u1[...]
        o = 0.5 * (num[:, :48] / den0[:, None] + num[:, 48:96] / den1[:, None]) \
            + b_ref[0][None, :]
        o_ref[...] = jnp.where(o > 0, o, jnp.exp(o) - 1.0)

    ub = pl.BlockSpec((RB, 128), lambda i: (i, 0))
    db = pl.BlockSpec((1, 1, RB), lambda i: (i, 0, 0))
    return pl.pallas_call(
        body,
        grid=(NB,),
        in_specs=[ub, ub, db, db, db, db,
                  pl.BlockSpec((1, 48), lambda i: (0, 0))],
        out_specs=pl.BlockSpec((RB, 48), lambda i: (i, 0)),
        out_shape=jax.ShapeDtypeStruct((N, 48), jnp.float32),
    )


@functools.lru_cache(maxsize=None)
def _make_post_self(cp):
    def body(u0, u1, d0, d1, b_ref, o_ref):
        den = d0[0, 0] + d1[0, 0] + 1e-16
        num = u0[...] + u1[...]
        o = num / den[:, None] + b_ref[0][None, :]
        o_ref[...] = jnp.where(o > 0, o, jnp.exp(o) - 1.0)

    ub = pl.BlockSpec((RB, cp), lambda i: (i, 0))
    db = pl.BlockSpec((1, 1, RB), lambda i: (i, 0, 0))
    return pl.pallas_call(
        body,
        grid=(NB,),
        in_specs=[ub, ub, db, db, pl.BlockSpec((1, cp), lambda i: (0, 0))],
        out_specs=ub,
        out_shape=jax.ShapeDtypeStruct((N, cp), jnp.float32),
    )


def _leaky(z):
    return jnp.maximum(z, 0.2 * z)


def _pad_rows(a):
    return jnp.pad(a, ((0, NP - a.shape[0]), (0, 0)))


def _prep_w(w, q, k, heads, outc, cp):
    d = w.shape[1]
    wp = w.reshape(R, d, heads, outc).transpose(2, 0, 1, 3)
    qp = q.reshape(R, heads, outc).transpose(1, 0, 2)[:, :, None, :]
    kp = k.reshape(R, heads, outc).transpose(1, 0, 2)[:, :, None, :]
    if cp != outc:
        pad = ((0, 0), (0, 0), (0, 0), (0, cp - outc))
        wp = jnp.pad(wp, pad)
        qp = jnp.pad(qp, pad)
        kp = jnp.pad(kp, pad)
    return wp, qp, kp


def _rgat(xp, w, q, k, b, src, dst, typ, outc, cp):
    wp, qp, kp = _prep_w(w, q, k, HEADS, outc, cp)
    xr, qt, kt, qm, km = _make_pre(HEADS, R, cp)(xp, wp, qp, kp)
    xrh = xr.reshape(HEADS, R * NP, cp)
    qtab = qt.reshape(HEADS, R * NP)
    ktab = kt.reshape(HEADS, R * NP)
    mh = _leaky(qm.reshape(HEADS, -1).max(axis=1) + km.reshape(HEADS, -1).max(axis=1))
    m160 = jnp.broadcast_to(jnp.reshape(mh[0], (1,)), (16,))
    m161 = jnp.broadcast_to(jnp.reshape(mh[1], (1,)), (16,))
    u0, u1, d0, d1 = _make_edge3(R * NP, NP)(
        qtab[0], qtab[1], ktab[0], ktab[1], src, dst, typ, xrh[0], xrh[1],
        m160, m161)
    bp = jnp.pad(b, (0, cp - b.shape[0]))[None, :]
    return _make_post_split(cp)(u0, u1, d0[:N].reshape(NB, 1, RB),
                                d1[:N].reshape(NB, 1, RB), bp)


def _rgat_pair(xp, w, q, k, b, src, dst, typ):
    d = w.shape[1]
    wp4 = jnp.zeros((R, d, 128), jnp.float32)
    wp4 = wp4.at[:, :, 0:OUTS].set(w.reshape(R, d, HEADS, OUTS)[:, :, 0])
    wp4 = wp4.at[:, :, 48:48 + OUTS].set(w.reshape(R, d, HEADS, OUTS)[:, :, 1])
    qp4 = jnp.zeros((HEADS, R, 128), jnp.float32)
    qp4 = qp4.at[0, :, 0:OUTS].set(q.reshape(R, HEADS, OUTS)[:, 0])
    qp4 = qp4.at[1, :, 48:48 + OUTS].set(q.reshape(R, HEADS, OUTS)[:, 1])
    kp4 = jnp.zeros((HEADS, R, 128), jnp.float32)
    kp4 = kp4.at[0, :, 0:OUTS].set(k.reshape(R, HEADS, OUTS)[:, 0])
    kp4 = kp4.at[1, :, 48:48 + OUTS].set(k.reshape(R, HEADS, OUTS)[:, 1])
    wp = jnp.broadcast_to(wp4[None], (HEADS, R, d, 128))
    xr, qt, kt, qm, km = _make_pre(HEADS, R, 128)(xp, wp, qp4[:, :, None, :], kp4[:, :, None, :])
    xrh = xr.reshape(HEADS, R * NP, 128)
    qtab = qt.reshape(HEADS, R * NP)
    ktab = kt.reshape(HEADS, R * NP)
    mh = _leaky(qm.reshape(HEADS, -1).max(axis=1) + km.reshape(HEADS, -1).max(axis=1))
    m0 = jnp.broadcast_to(jnp.reshape(mh[0], (1,)), (16,))
    m1 = jnp.broadcast_to(jnp.reshape(mh[1], (1,)), (16,))
    u0, u1, d00, d01, d10, d11 = _make_edge2(R * NP, NP)(
        qtab[0], qtab[1], ktab[0], ktab[1], src, dst, typ, xrh[0], m0, m1)
    bp = jnp.pad(b, (0, 48 - OUTS))[None, :]
    return _make_post_pair()(
        u0, u1,
        d00[:N].reshape(NB, 1, RB), d01[:N].reshape(NB, 1, RB),
        d10[:N].reshape(NB, 1, RB), d11[:N].reshape(NB, 1, RB), bp)


def kernel(x, edge_index, edge_type, W1, Q1, K1, b1, WS, QS, KS, bS,
           W2, Q2, K2, b2, WT, QT, KT, bT, Wr, a_src, a_dst, br):
    src = edge_index[0]
    dst = edge_index[1]
    typ = edge_type
    xp = _pad_rows(x)

    x_s = _rgat(xp, W1, Q1, K1, b1, src, dst, typ, HID, HID)
    x_t = _rgat(xp, W2, Q2, K2, b2, src, dst, typ, HID, HID)
    x_in = _rgat_pair(_pad_rows(x_s), WS, QS, KS, bS, src, dst, typ)[:, :OUTS]
    x_out = _rgat_pair(_pad_rows(x_t), WT, QT, KT, bT, src, dst, typ)[:, :OUTS]

    wr = Wr[None, None]
    qp = a_dst[None, None, None, :]
    kp = a_src[None, None, None, :]
    xr, qt, kt, qm, km = _make_pre(1, 1, HID)(xp, wr, qp, kp)
    m0 = _leaky(qm.max() + km.max())
    m16 = jnp.broadcast_to(jnp.reshape(m0, (1,)), (16,))
    u0, u1, d0, d1 = _make_edge(NP, HID, 0)(
        qt.reshape(NP), kt.reshape(NP), src, dst, typ, xr.reshape(NP, HID), m16)
    x_self = _make_post_self(HID)(u0, u1, d0[:N].reshape(NB, 1, RB), d1[:N].reshape(NB, 1, RB),
                                  br[None, :])
    return (x_in, x_out, x_self)
